# Initial kernel scaffold; baseline (speedup 1.0000x reference)
import jax
import jax.numpy as jnp
from jax.experimental import pallas as pl


def kernel(node_features, edge_index, root_index, batch_size, params, noise):
    raise NotImplementedError("write your pallas kernel here")


# speedup vs baseline: 1.0001x; 1.0001x over previous
import functools

import jax
import jax.numpy as jnp
import numpy as np
from jax import lax
from jax.experimental import pallas as pl
from jax.experimental.pallas import tpu as pltpu

N = 10000
E = 160000
D_IN = 128
H = 16
D_OUT = 128
EDGE_NUM = 4
G = 64
EPS = 1e-5


def _passthrough_body(x_ref, o_ref):
    o_ref[...] = x_ref[...]


def _passthrough(x):
    return pl.pallas_call(
        _passthrough_body,
        out_shape=jax.ShapeDtypeStruct(x.shape, x.dtype),
    )(x)


def _apply_net(p, x):
    y = jnp.einsum('oc,ecl->eol', p['w1'], x)
    m = y.mean(axis=(0, 2), keepdims=True)
    v = y.var(axis=(0, 2), keepdims=True)
    y = (y - m) / jnp.sqrt(v + EPS) * p['g'][None, :, None] + p['b'][None, :, None]
    y = jnp.where(y > 0, y, 0.01 * y)
    y = jnp.einsum('oc,ecl->eol', p['w2'], y) + p['b2'][None, :, None]
    return y


def _gcn_conv(x, edge_index, W, b, edge_weight=None):
    n = x.shape[0]
    row, col = edge_index[0], edge_index[1]
    if edge_weight is None:
        ew = jnp.ones((row.shape[0],), dtype=jnp.float32)
    else:
        ew = edge_weight
    loop = jnp.arange(n, dtype=row.dtype)
    row = jnp.concatenate([row, loop])
    col = jnp.concatenate([col, loop])
    ew = jnp.concatenate([ew, jnp.ones((n,), dtype=jnp.float32)])
    deg = jnp.zeros((n,), dtype=jnp.float32).at[col].add(ew)
    dis = jnp.where(deg > 0, deg ** -0.5, 0.0)
    norm = dis[row] * ew * dis[col]
    xw = x @ W
    msg = jnp.take(xw, row, axis=0) * norm[:, None]
    out = jnp.zeros((n, W.shape[1]), dtype=jnp.float32).at[col].add(msg)
    return out + b


def _edge_infer(params, x, edge_index, noise):
    n = x.shape[0]
    row, col = edge_index[0], edge_index[1]
    xi = jnp.take(x, (row - 1) % n, axis=0)[:, :, None]
    xj = jnp.take(x, (col - 1) % n, axis=0)[:, None, :]
    x_ij = jnp.abs(xi - xj)
    sim_val = _apply_net(params['sim'], x_ij)
    edge_pred = jax.nn.sigmoid(jnp.einsum('eol,kl->eok', sim_val, params['fc1']))
    w_mean = _apply_net(params['wm'], x_ij)
    w_bias = _apply_net(params['wb'], x_ij)
    b_mean = _apply_net(params['bm'], x_ij)
    b_bias = _apply_net(params['bb'], x_ij)
    logit_mean = w_mean * sim_val + b_mean
    logit_var = jnp.abs(jnp.log(sim_val ** 2 * jnp.exp(w_bias) + jnp.exp(b_bias)))
    edge_y = jax.nn.sigmoid(logit_mean + logit_var * noise)
    edge_y = jnp.einsum('eol,kl->eok', edge_y, params['fc2'])
    logp_x = jax.nn.log_softmax(edge_pred, axis=-1)
    p_y = jax.nn.softmax(edge_y, axis=-1)
    unsup = jnp.sum(p_y * (jnp.log(p_y) - logp_x)) / edge_pred.shape[0]
    ep = jnp.mean(edge_pred, axis=-1)[:, 0]
    return unsup, ep


def kernel(node_features, edge_index, root_index, batch_size, params, noise):
    x0 = _passthrough(node_features)
    h1 = _gcn_conv(x0, edge_index, params['W1'], params['b1'])
    edge_loss, edge_pred = _edge_infer(params, h1, edge_index, noise)
    root_ext = jnp.take(x0, jnp.take(root_index, batch_size), axis=0)
    x = jnp.concatenate([h1, root_ext], axis=1)
    m = x.mean(axis=0)
    v = x.var(axis=0)
    x = (x - m) / jnp.sqrt(v + EPS) * params['bn1_g'] + params['bn1_b']
    x = jax.nn.relu(x)
    x = _gcn_conv(x, edge_index, params['W2'], params['b2'], edge_pred)
    x = jax.nn.relu(x)
    root_ext2 = jnp.take(h1, jnp.take(root_index, batch_size), axis=0)
    x = jnp.concatenate([x, root_ext2], axis=1)
    sums = jax.ops.segment_sum(x, batch_size, num_segments=G)
    cnt = jax.ops.segment_sum(jnp.ones((x.shape[0],), dtype=jnp.float32), batch_size, num_segments=G)
    out = sums / jnp.clip(cnt, 1.0)[:, None]
    return out, edge_loss


# speedup vs baseline: 5.5030x; 5.5023x over previous
import functools

import jax
import jax.numpy as jnp
import numpy as np
from jax import lax
from jax.experimental import pallas as pl
from jax.experimental.pallas import tpu as pltpu

N = 10000
E = 160000
D_IN = 128
H = 16
G = 64
EPS = 1e-5

_INTERP = False

EB = 640
NEB = E // EB
RB = 1000
NRB = N // RB


def _xw1_body(x0_ref, w1_ref, p0_ref, p1_ref, xws_ref, dis_ref):
    dis = lax.rsqrt(p0_ref[...] + p1_ref[...] + 1.0)
    xw = jnp.dot(x0_ref[...], w1_ref[...], preferred_element_type=jnp.float32)
    xws_ref[...] = xw * dis
    dis_ref[...] = dis


def _tc_xw1(x0, W1, p0, p1):
    return pl.pallas_call(
        _xw1_body,
        grid=(NRB,),
        in_specs=[
            pl.BlockSpec((RB, D_IN), lambda i: (i, 0)),
            pl.BlockSpec((D_IN, H), lambda i: (0, 0)),
            pl.BlockSpec((RB, H), lambda i: (i, 0)),
            pl.BlockSpec((RB, H), lambda i: (i, 0)),
        ],
        out_specs=[
            pl.BlockSpec((RB, H), lambda i: (i, 0)),
            pl.BlockSpec((RB, H), lambda i: (i, 0)),
        ],
        out_shape=[
            jax.ShapeDtypeStruct((N, H), jnp.float32),
            jax.ShapeDtypeStruct((N, H), jnp.float32),
        ],
        interpret=_INTERP,
    )(x0, W1, p0, p1)


def _h1_body(p0_ref, p1_ref, xws_ref, dis_ref, b_ref, o_ref):
    o_ref[...] = dis_ref[...] * (p0_ref[...] + p1_ref[...] + xws_ref[...]) + b_ref[...]


def _tc_h1(p0, p1, xws1, dis1, b1):
    return pl.pallas_call(
        _h1_body,
        grid=(NRB,),
        in_specs=[
            pl.BlockSpec((RB, H), lambda i: (i, 0)),
            pl.BlockSpec((RB, H), lambda i: (i, 0)),
            pl.BlockSpec((RB, H), lambda i: (i, 0)),
            pl.BlockSpec((RB, H), lambda i: (i, 0)),
            pl.BlockSpec((1, H), lambda i: (0, 0)),
        ],
        out_specs=pl.BlockSpec((RB, H), lambda i: (i, 0)),
        out_shape=jax.ShapeDtypeStruct((N, H), jnp.float32),
        interpret=_INTERP,
    )(p0, p1, xws1, dis1, b1.reshape(1, H))


def _moments_body(xiT_ref, xjT_ref, s_ref, m_ref):
    @pl.when(pl.program_id(0) == 0)
    def _init():
        s_ref[...] = jnp.zeros_like(s_ref)
        m_ref[...] = jnp.zeros_like(m_ref)

    xiT = xiT_ref[...]
    xjT = xjT_ref[...]
    s = jnp.zeros((H, H), jnp.float32)
    m = jnp.zeros((H, 1), jnp.float32)
    for l in range(H):
        z = jnp.abs(xiT - xjT[l:l + 1, :])
        s = s + lax.dot_general(z, z, (((1,), (1,)), ((), ())),
                                preferred_element_type=jnp.float32)
        m = m + jnp.sum(z, axis=1, keepdims=True)
    s_ref[...] += s
    m_ref[...] += m


def _tc_moments(xiT, xjT):
    return pl.pallas_call(
        _moments_body,
        grid=(NEB,),
        in_specs=[
            pl.BlockSpec((H, EB), lambda i: (0, i)),
            pl.BlockSpec((H, EB), lambda i: (0, i)),
        ],
        out_specs=[
            pl.BlockSpec((H, H), lambda i: (0, 0)),
            pl.BlockSpec((H, 1), lambda i: (0, 0)),
        ],
        out_shape=[
            jax.ShapeDtypeStruct((H, H), jnp.float32),
            jax.ShapeDtypeStruct((H, 1), jnp.float32),
        ],
        interpret=_INTERP,
    )(xiT, xjT)


def _logsumexp0(x):
    mx = jnp.max(x, axis=0, keepdims=True)
    return jnp.log(jnp.sum(jnp.exp(x - mx), axis=0, keepdims=True)) + mx


def _pass2_body(xiT_ref, xjT_ref, nzT_ref, w1e_ref, be_ref, w2b_ref, b2s_ref,
                fc1_ref, fc2_ref, ep_ref, kl_ref, sim_s, wm_s, wb_s, bm_s, bb_s):
    @pl.when(pl.program_id(0) == 0)
    def _init():
        kl_ref[...] = jnp.zeros_like(kl_ref)

    xiT = xiT_ref[...]
    xjT = xjT_ref[...]
    w1e = w1e_ref[...]
    be = be_ref[...]
    w2b = w2b_ref[...]
    b2s = b2s_ref[...]
    for l in range(H):
        z = jnp.abs(xiT - xjT[l:l + 1, :])
        y = jnp.dot(w1e, z, preferred_element_type=jnp.float32) + be
        y = jnp.where(y > 0, y, 0.01 * y)
        s5 = jnp.dot(w2b, y, preferred_element_type=jnp.float32) + b2s
        sim_s[l:l + 1, :] = s5[0:1, :]
        wm_s[l:l + 1, :] = s5[1:2, :]
        wb_s[l:l + 1, :] = s5[2:3, :]
        bm_s[l:l + 1, :] = s5[3:4, :]
        bb_s[l:l + 1, :] = s5[4:5, :]
    sv = sim_s[...]
    ep_logits = jnp.dot(fc1_ref[...], sv, preferred_element_type=jnp.float32)
    edge_pred = jax.nn.sigmoid(ep_logits)
    lm = wm_s[...] * sv + bm_s[...]
    lv = jnp.abs(jnp.log(sv * sv * jnp.exp(wb_s[...]) + jnp.exp(bb_s[...])))
    ey_in = jax.nn.sigmoid(lm + lv * nzT_ref[...])
    edge_y = jnp.dot(fc2_ref[...], ey_in, preferred_element_type=jnp.float32)
    logp_x = edge_pred - _logsumexp0(edge_pred)
    logp_y = edge_y - _logsumexp0(edge_y)
    p_y = jnp.exp(logp_y)
    kl_blk = jnp.sum(jnp.sum(p_y * (logp_y - logp_x), axis=1, keepdims=True),
                     axis=0, keepdims=True)
    kl_ref[...] += kl_blk
    ep_ref[...] = jnp.mean(edge_pred, axis=0, keepdims=True)[None]


def _tc_pass2(xiT, xjT, noiseT, W1eff, beff, W2blk, b2s, fc1, fc2):
    return pl.pallas_call(
        _pass2_body,
        grid=(NEB,),
        in_specs=[
            pl.BlockSpec((H, EB), lambda i: (0, i)),
            pl.BlockSpec((H, EB), lambda i: (0, i)),
            pl.BlockSpec((H, EB), lambda i: (0, i)),
            pl.BlockSpec((80, H), lambda i: (0, 0)),
            pl.BlockSpec((80, 1), lambda i: (0, 0)),
            pl.BlockSpec((5, 80), lambda i: (0, 0)),
            pl.BlockSpec((5, 1), lambda i: (0, 0)),
            pl.BlockSpec((4, H), lambda i: (0, 0)),
            pl.BlockSpec((4, H), lambda i: (0, 0)),
        ],
        out_specs=[
            pl.BlockSpec((1, 1, EB), lambda i: (i, 0, 0)),
            pl.BlockSpec((1, 1), lambda i: (0, 0)),
        ],
        out_shape=[
            jax.ShapeDtypeStruct((NEB, 1, EB), jnp.float32),
            jax.ShapeDtypeStruct((1, 1), jnp.float32),
        ],
        scratch_shapes=[pltpu.VMEM((H, EB), jnp.float32) for _ in range(5)],
        interpret=_INTERP,
    )(xiT, xjT, noiseT, W1eff, beff, W2blk, b2s, fc1, fc2)


def _bnstat_body(h1_ref, re_ref, s_ref, q_ref):
    @pl.when(pl.program_id(0) == 0)
    def _init():
        s_ref[...] = jnp.zeros_like(s_ref)
        q_ref[...] = jnp.zeros_like(q_ref)

    x = jnp.concatenate([h1_ref[...], re_ref[...]], axis=1)
    s_ref[...] += jnp.sum(x, axis=0, keepdims=True)
    q_ref[...] += jnp.sum(x * x, axis=0, keepdims=True)


def _tc_bnstat(h1, root_ext):
    return pl.pallas_call(
        _bnstat_body,
        grid=(NRB,),
        in_specs=[
            pl.BlockSpec((RB, H), lambda i: (i, 0)),
            pl.BlockSpec((RB, D_IN), lambda i: (i, 0)),
        ],
        out_specs=[
            pl.BlockSpec((1, H + D_IN), lambda i: (0, 0)),
            pl.BlockSpec((1, H + D_IN), lambda i: (0, 0)),
        ],
        out_shape=[
            jax.ShapeDtypeStruct((1, H + D_IN), jnp.float32),
            jax.ShapeDtypeStruct((1, H + D_IN), jnp.float32),
        ],
        interpret=_INTERP,
    )(h1, root_ext)


def _xw2_body(h1_ref, re_ref, sc_ref, sh_ref, w2_ref, p0_ref, p1_ref,
              xws_ref, dis_ref):
    dis = lax.rsqrt(p0_ref[...] + p1_ref[...] + 1.0)
    x = jnp.concatenate([h1_ref[...], re_ref[...]], axis=1)
    x = jax.nn.relu(x * sc_ref[...] + sh_ref[...])
    xw = jnp.dot(x, w2_ref[...], preferred_element_type=jnp.float32)
    xws_ref[...] = xw * dis[:, 0:1]
    dis_ref[...] = dis


def _tc_xw2(h1, root_ext, scale, shift, W2, p0, p1):
    return pl.pallas_call(
        _xw2_body,
        grid=(NRB,),
        in_specs=[
            pl.BlockSpec((RB, H), lambda i: (i, 0)),
            pl.BlockSpec((RB, D_IN), lambda i: (i, 0)),
            pl.BlockSpec((1, H + D_IN), lambda i: (0, 0)),
            pl.BlockSpec((1, H + D_IN), lambda i: (0, 0)),
            pl.BlockSpec((H + D_IN, D_IN), lambda i: (0, 0)),
            pl.BlockSpec((RB, H), lambda i: (i, 0)),
            pl.BlockSpec((RB, H), lambda i: (i, 0)),
        ],
        out_specs=[
            pl.BlockSpec((RB, D_IN), lambda i: (i, 0)),
            pl.BlockSpec((RB, H), lambda i: (i, 0)),
        ],
        out_shape=[
            jax.ShapeDtypeStruct((N, D_IN), jnp.float32),
            jax.ShapeDtypeStruct((N, H), jnp.float32),
        ],
        interpret=_INTERP,
    )(h1, root_ext, scale, shift, W2, p0, p1)


def _out_body(p0a_ref, p1a_ref, p0b_ref, p1b_ref, xws_ref, dis_ref, b2_ref,
              re2_ref, bat_ref, out_ref, sums, cnts):
    @pl.when(pl.program_id(0) == 0)
    def _init():
        sums[...] = jnp.zeros_like(sums)
        cnts[...] = jnp.zeros_like(cnts)

    psum = jnp.concatenate([p0a_ref[...] + p1a_ref[...],
                            p0b_ref[...] + p1b_ref[...]], axis=1)
    x2 = jax.nn.relu(dis_ref[...][:, 0:1]
                     * (psum + xws_ref[...]) + b2_ref[...])
    z = jnp.concatenate([x2, re2_ref[...]], axis=1)
    bat = bat_ref[0]
    oh = (lax.broadcasted_iota(jnp.int32, (G, RB), 0) == bat).astype(jnp.float32)
    sums[...] += jnp.dot(oh, z, preferred_element_type=jnp.float32)
    cnts[...] += jnp.sum(oh, axis=1, keepdims=True)

    @pl.when(pl.program_id(0) == NRB - 1)
    def _fin():
        out_ref[...] = sums[...] / jnp.maximum(cnts[...], 1.0)


def _tc_out(p0a, p1a, p0b, p1b, xws2, dis2, b2, root_ext2, batch3):
    HD = D_IN // 2
    return pl.pallas_call(
        _out_body,
        grid=(NRB,),
        in_specs=[
            pl.BlockSpec((RB, HD), lambda i: (i, 0)),
            pl.BlockSpec((RB, HD), lambda i: (i, 0)),
            pl.BlockSpec((RB, HD), lambda i: (i, 0)),
            pl.BlockSpec((RB, HD), lambda i: (i, 0)),
            pl.BlockSpec((RB, D_IN), lambda i: (i, 0)),
            pl.BlockSpec((RB, H), lambda i: (i, 0)),
            pl.BlockSpec((1, D_IN), lambda i: (0, 0)),
            pl.BlockSpec((RB, H), lambda i: (i, 0)),
            pl.BlockSpec((1, 1, RB), lambda i: (i, 0, 0)),
        ],
        out_specs=pl.BlockSpec((G, H + D_IN), lambda i: (0, 0)),
        out_shape=jax.ShapeDtypeStruct((G, H + D_IN), jnp.float32),
        scratch_shapes=[
            pltpu.VMEM((G, H + D_IN), jnp.float32),
            pltpu.VMEM((G, 1), jnp.float32),
        ],
        interpret=_INTERP,
    )(p0a, p1a, p0b, p1b, xws2, dis2, b2.reshape(1, D_IN), root_ext2, batch3)


from jax.experimental.pallas import tpu_sc as plsc

NC = 2
NS = 16
NW = NC * NS
CH = 128
E_PAD = 163840
EPT = E_PAD // NW
NCHUNK = EPT // CH
N_PAD = 10240
ZPT = N_PAD // NS
NPT = N_PAD // NW
RCH = 64


def _sc_mesh():
    return plsc.VectorSubcoreMesh(core_axis_name="c", subcore_axis_name="s")


_SC_PARAMS = dict(compiler_params=pltpu.CompilerParams(use_tc_tiling_on_sc=False))


def _zero_vmem_rows(ref, nrows, width):
    def zrow(r, _):
        for j in range(width // 16):
            ref[r, pl.ds(16 * j, 16)] = jnp.zeros((16,), jnp.float32)
        return 0
    lax.fori_loop(0, nrows, zrow, 0)


def _sc_hist(col_pad, wrows_pad):
    @functools.partial(
        pl.kernel, mesh=_sc_mesh(), **_SC_PARAMS,
        out_type=[
            jax.ShapeDtypeStruct((N_PAD, H), jnp.float32),
            jax.ShapeDtypeStruct((N_PAD, H), jnp.float32),
        ],
        scratch_types=[
            pltpu.VMEM((CH,), jnp.int32),
            pltpu.VMEM((CH, H), jnp.float32),
            pltpu.VMEM((ZPT, H), jnp.float32),
            pltpu.VMEM_SHARED((N_PAD, H), jnp.float32),
            pltpu.SemaphoreType.DMA,
        ],
    )
    def k(col_hbm, w_hbm, out0, out1, cidx_v, rows_v, ztile_v, accum, sem):
        cid = lax.axis_index("c")
        sid = lax.axis_index("s")
        wid = cid * NS + sid
        _zero_vmem_rows(ztile_v, ZPT, H)
        pltpu.sync_copy(ztile_v, accum.at[pl.ds(sid * ZPT, ZPT)])
        plsc.subcore_barrier()

        def chunk(i, _):
            base = wid * EPT + i * CH
            pltpu.sync_copy(col_hbm.at[pl.ds(base, CH)], cidx_v)
            pltpu.sync_copy(w_hbm.at[pl.ds(base, CH)], rows_v)
            pltpu.sync_copy(rows_v, accum.at[cidx_v], add=True)
            return 0
        lax.fori_loop(0, NCHUNK, chunk, 0)
        plsc.subcore_barrier()

        @pl.when(cid == 0)
        def _c0():
            pltpu.sync_copy(accum.at[pl.ds(sid * ZPT, ZPT)],
                            out0.at[pl.ds(sid * ZPT, ZPT)])

        @pl.when(cid == 1)
        def _c1():
            pltpu.sync_copy(accum.at[pl.ds(sid * ZPT, ZPT)],
                            out1.at[pl.ds(sid * ZPT, ZPT)])

    return k(col_pad, wrows_pad)


def _sc_conv_scatter(table, row_pad, col_pad, D, ewrows=None):
    has_w = ewrows is not None

    @functools.partial(
        pl.kernel, mesh=_sc_mesh(), **_SC_PARAMS,
        out_type=[
            jax.ShapeDtypeStruct((N_PAD, D), jnp.float32),
            jax.ShapeDtypeStruct((N_PAD, D), jnp.float32),
        ],
        scratch_types=[
            pltpu.VMEM((CH,), jnp.int32),
            pltpu.VMEM((CH,), jnp.int32),
            pltpu.VMEM((CH, H), jnp.float32),
            pltpu.VMEM((CH, D), jnp.float32),
            pltpu.VMEM((ZPT, D), jnp.float32),
            pltpu.VMEM_SHARED((N_PAD, D), jnp.float32),
            pltpu.SemaphoreType.DMA,
        ],
    )
    def k(table_hbm, row_hbm, col_hbm, w_hbm, out0, out1,
          ridx_v, cidx_v, w_v, rows_v, ztile_v, accum, sem):
        cid = lax.axis_index("c")
        sid = lax.axis_index("s")
        wid = cid * NS + sid
        _zero_vmem_rows(ztile_v, ZPT, D)
        pltpu.sync_copy(ztile_v, accum.at[pl.ds(sid * ZPT, ZPT)])
        plsc.subcore_barrier()

        def chunk(i, _):
            base = wid * EPT + i * CH
            pltpu.sync_copy(row_hbm.at[pl.ds(base, CH)], ridx_v)
            pltpu.sync_copy(col_hbm.at[pl.ds(base, CH)], cidx_v)
            pltpu.async_copy(table_hbm.at[ridx_v], rows_v, sem).wait()
            if has_w:
                pltpu.sync_copy(w_hbm.at[pl.ds(base, CH)], w_v)

                def scale(e, _):
                    bc = w_v[e, :]
                    for j in range(D // 16):
                        s = pl.ds(16 * j, 16)
                        rows_v[e, s] = rows_v[e, s] * bc
                    return 0
                lax.fori_loop(0, CH, scale, 0)
            pltpu.sync_copy(rows_v, accum.at[cidx_v], add=True)
            return 0
        lax.fori_loop(0, NCHUNK, chunk, 0)
        plsc.subcore_barrier()

        @pl.when(cid == 0)
        def _c0():
            pltpu.sync_copy(accum.at[pl.ds(sid * ZPT, ZPT)],
                            out0.at[pl.ds(sid * ZPT, ZPT)])

        @pl.when(cid == 1)
        def _c1():
            pltpu.sync_copy(accum.at[pl.ds(sid * ZPT, ZPT)],
                            out1.at[pl.ds(sid * ZPT, ZPT)])

    if not has_w:
        ewrows = jnp.zeros((E_PAD, H), jnp.float32)
    return k(table, row_pad, col_pad, ewrows)


def _sc_edge_gather(h1, row_pad, col_pad):
    @functools.partial(
        pl.kernel, mesh=_sc_mesh(), **_SC_PARAMS,
        out_type=[
            jax.ShapeDtypeStruct((E_PAD, H), jnp.float32),
            jax.ShapeDtypeStruct((E_PAD, H), jnp.float32),
        ],
        scratch_types=[
            pltpu.VMEM((CH,), jnp.int32),
            pltpu.VMEM((CH,), jnp.int32),
            pltpu.VMEM((CH, H), jnp.float32),
            pltpu.SemaphoreType.DMA,
        ],
    )
    def k(h1_hbm, row_hbm, col_hbm, xi_hbm, xj_hbm, raw_v, idx_v, rows_v, sem):
        cid = lax.axis_index("c")
        sid = lax.axis_index("s")
        wid = cid * NS + sid

        def chunk(i, _):
            base = wid * EPT + i * CH
            for src_hbm, dst_hbm in ((row_hbm, xi_hbm), (col_hbm, xj_hbm)):
                pltpu.sync_copy(src_hbm.at[pl.ds(base, CH)], raw_v)
                for t in range(CH // 16):
                    s = pl.ds(16 * t, 16)
                    v = raw_v[s] - 1
                    idx_v[s] = jnp.where(v < 0, v + N, v)
                pltpu.async_copy(h1_hbm.at[idx_v], rows_v, sem).wait()
                pltpu.sync_copy(rows_v, dst_hbm.at[pl.ds(base, CH)])
            return 0
        lax.fori_loop(0, NCHUNK, chunk, 0)

    return k(h1, row_pad, col_pad)


def _sc_root_gather(x0, h1, root_index, batch_pad):
    @functools.partial(
        pl.kernel, mesh=_sc_mesh(), **_SC_PARAMS,
        out_type=[
            jax.ShapeDtypeStruct((N_PAD, D_IN), jnp.float32),
            jax.ShapeDtypeStruct((N_PAD, H), jnp.float32),
        ],
        scratch_types=[
            pltpu.VMEM((RCH,), jnp.int32),
            pltpu.VMEM((RCH,), jnp.int32),
            pltpu.VMEM((RCH, D_IN), jnp.float32),
            pltpu.VMEM((RCH, H), jnp.float32),
            pltpu.SemaphoreType.DMA,
        ],
    )
    def k(x0_hbm, h1_hbm, ri_hbm, bat_hbm, re_hbm, re2_hbm,
          braw_v, idx_v, rows1_v, rows2_v, sem):
        cid = lax.axis_index("c")
        sid = lax.axis_index("s")
        wid = cid * NS + sid

        def chunk(i, _):
            base = wid * NPT + i * RCH
            pltpu.sync_copy(bat_hbm.at[pl.ds(base, RCH)], braw_v)
            pltpu.async_copy(ri_hbm.at[braw_v], idx_v, sem).wait()
            pltpu.async_copy(x0_hbm.at[idx_v], rows1_v, sem).wait()
            pltpu.sync_copy(rows1_v, re_hbm.at[pl.ds(base, RCH)])
            pltpu.async_copy(h1_hbm.at[idx_v], rows2_v, sem).wait()
            pltpu.sync_copy(rows2_v, re2_hbm.at[pl.ds(base, RCH)])
            return 0
        lax.fori_loop(0, NPT // RCH, chunk, 0)

    return k(x0, h1, root_index, batch_pad)


def kernel(node_features, edge_index, root_index, batch_size, params, noise):
    x0 = node_features
    row, col = edge_index[0], edge_index[1]

    npad = E_PAD - E
    pad_src = (jnp.arange(npad, dtype=jnp.int32) % 240)
    pad_dst = N + (jnp.arange(npad, dtype=jnp.int32) % (N_PAD - N))
    row_pad = jnp.concatenate([row, pad_src])
    col_pad = jnp.concatenate([col, pad_dst])
    zpad_e = jnp.zeros((npad,), jnp.float32)
    batch_pad = jnp.concatenate(
        [batch_size, jnp.zeros((N_PAD - N,), jnp.int32)])

    ones_rows = jnp.concatenate(
        [jnp.ones((E, H), jnp.float32), jnp.zeros((E_PAD - E, H), jnp.float32)])
    d1p0, d1p1 = _sc_hist(col_pad, ones_rows)
    xws1, dis1 = _tc_xw1(x0, params['W1'], d1p0, d1p1)

    c1p0, c1p1 = _sc_conv_scatter(xws1, row_pad, col_pad, H)
    h1 = _tc_h1(c1p0, c1p1, xws1, dis1, params['b1'])

    xi, xj = _sc_edge_gather(h1, row_pad, col_pad)
    root_ext, root_ext2 = _sc_root_gather(x0, h1, root_index, batch_pad)

    xiT = xi[:E].T
    xjT = xj[:E].T
    noiseT = noise[:, 0, :].T

    Ssum, Msum = _tc_moments(xiT, xjT)
    cntm = float(E * H)
    Mm = Msum[:, 0] / cntm
    Sm = Ssum / cntm
    w1e_rows, be_rows, w2b_rows, b2s_rows = [], [], [], []
    for k, name in enumerate(('sim', 'wm', 'wb', 'bm', 'bb')):
        p = params[name]
        w1 = p['w1']
        mean = w1 @ Mm
        ey2 = jnp.einsum('oc,cd,od->o', w1, Sm, w1)
        var = ey2 - mean * mean
        sc = p['g'] * lax.rsqrt(var + EPS)
        w1e_rows.append(w1 * sc[:, None])
        be_rows.append(p['b'] - mean * sc)
        wrow = jnp.zeros((5, 80), jnp.float32).at[k, 16 * k:16 * k + 16].set(p['w2'][0])
        w2b_rows.append(wrow)
        b2s_rows.append(p['b2'][0])
    W1eff = jnp.concatenate(w1e_rows, axis=0)
    beff = jnp.concatenate(be_rows).reshape(80, 1)
    W2blk = sum(w2b_rows)
    b2s = jnp.stack(b2s_rows).reshape(5, 1)

    ep3, klsum = _tc_pass2(xiT, xjT, noiseT, W1eff, beff, W2blk, b2s,
                           params['fc1'], params['fc2'])
    ep = ep3.reshape(E)
    edge_loss = klsum[0, 0] / float(E)

    s1, q1 = _tc_bnstat(h1, root_ext)
    mb = s1 / float(N)
    vb = q1 / float(N) - mb * mb
    scb = params['bn1_g'].reshape(1, -1) * lax.rsqrt(vb + EPS)
    shb = params['bn1_b'].reshape(1, -1) - mb * scb

    ep_rows = jnp.concatenate(
        [jnp.broadcast_to(ep[:, None], (E, H)),
         jnp.zeros((E_PAD - E, H), jnp.float32)])
    d2p0, d2p1 = _sc_hist(col_pad, ep_rows)
    xws2, dis2 = _tc_xw2(h1, root_ext, scb, shb, params['W2'], d2p0, d2p1)
    HD = D_IN // 2
    c2p0a, c2p1a = _sc_conv_scatter(xws2[:, :HD], row_pad, col_pad, HD,
                                    ewrows=ep_rows)
    c2p0b, c2p1b = _sc_conv_scatter(xws2[:, HD:], row_pad, col_pad, HD,
                                    ewrows=ep_rows)

    batch3 = batch_size.reshape(NRB, 1, RB)
    out = _tc_out(c2p0a, c2p1a, c2p0b, c2p1b, xws2, dis2, params['b2'],
                  root_ext2, batch3)
    return out, edge_loss


# speedup vs baseline: 7.1840x; 1.3055x over previous
import functools

import jax
import jax.numpy as jnp
import numpy as np
from jax import lax
from jax.experimental import pallas as pl
from jax.experimental.pallas import tpu as pltpu

N = 10000
E = 160000
D_IN = 128
H = 16
G = 64
EPS = 1e-5

_INTERP = False

EB = 640
NEB = E // EB
RB = 1000
NRB = N // RB


def _xw1_body(x0_ref, w1_ref, p0_ref, p1_ref, xws_ref, dis_ref):
    dis = lax.rsqrt(p0_ref[...] + p1_ref[...] + 1.0)
    xw = jnp.dot(x0_ref[...], w1_ref[...], preferred_element_type=jnp.float32)
    xws_ref[...] = xw * dis
    dis_ref[...] = dis


def _tc_xw1(x0, W1, p0, p1):
    return pl.pallas_call(
        _xw1_body,
        grid=(NRB,),
        in_specs=[
            pl.BlockSpec((RB, D_IN), lambda i: (i, 0)),
            pl.BlockSpec((D_IN, H), lambda i: (0, 0)),
            pl.BlockSpec((RB, H), lambda i: (i, 0)),
            pl.BlockSpec((RB, H), lambda i: (i, 0)),
        ],
        out_specs=[
            pl.BlockSpec((RB, H), lambda i: (i, 0)),
            pl.BlockSpec((RB, H), lambda i: (i, 0)),
        ],
        out_shape=[
            jax.ShapeDtypeStruct((N, H), jnp.float32),
            jax.ShapeDtypeStruct((N, H), jnp.float32),
        ],
        interpret=_INTERP,
    )(x0, W1, p0, p1)


def _h1_body(p0_ref, p1_ref, xws_ref, dis_ref, b_ref, o_ref):
    o_ref[...] = dis_ref[...] * (p0_ref[...] + p1_ref[...] + xws_ref[...]) + b_ref[...]


def _tc_h1(p0, p1, xws1, dis1, b1):
    return pl.pallas_call(
        _h1_body,
        grid=(NRB,),
        in_specs=[
            pl.BlockSpec((RB, H), lambda i: (i, 0)),
            pl.BlockSpec((RB, H), lambda i: (i, 0)),
            pl.BlockSpec((RB, H), lambda i: (i, 0)),
            pl.BlockSpec((RB, H), lambda i: (i, 0)),
            pl.BlockSpec((1, H), lambda i: (0, 0)),
        ],
        out_specs=pl.BlockSpec((RB, H), lambda i: (i, 0)),
        out_shape=jax.ShapeDtypeStruct((N, H), jnp.float32),
        interpret=_INTERP,
    )(p0, p1, xws1, dis1, b1.reshape(1, H))


def _moments_body(xiT_ref, xjT_ref, s_ref, m_ref):
    @pl.when(pl.program_id(0) == 0)
    def _init():
        s_ref[...] = jnp.zeros_like(s_ref)
        m_ref[...] = jnp.zeros_like(m_ref)

    xiT = xiT_ref[...]
    xjT = xjT_ref[...]
    s = jnp.zeros((H, H), jnp.float32)
    m = jnp.zeros((H, 1), jnp.float32)
    for l in range(H):
        z = jnp.abs(xiT - xjT[l:l + 1, :])
        s = s + lax.dot_general(z, z, (((1,), (1,)), ((), ())),
                                preferred_element_type=jnp.float32)
        m = m + jnp.sum(z, axis=1, keepdims=True)
    s_ref[...] += s
    m_ref[...] += m


def _tc_moments(xiT, xjT):
    return pl.pallas_call(
        _moments_body,
        grid=(NEB,),
        in_specs=[
            pl.BlockSpec((H, EB), lambda i: (0, i)),
            pl.BlockSpec((H, EB), lambda i: (0, i)),
        ],
        out_specs=[
            pl.BlockSpec((H, H), lambda i: (0, 0)),
            pl.BlockSpec((H, 1), lambda i: (0, 0)),
        ],
        out_shape=[
            jax.ShapeDtypeStruct((H, H), jnp.float32),
            jax.ShapeDtypeStruct((H, 1), jnp.float32),
        ],
        interpret=_INTERP,
    )(xiT, xjT)


def _logsumexp0(x):
    mx = jnp.max(x, axis=0, keepdims=True)
    return jnp.log(jnp.sum(jnp.exp(x - mx), axis=0, keepdims=True)) + mx


def _pass2_body(xiT_ref, xjT_ref, nzT_ref, w1e_ref, be_ref, w2b_ref, b2s_ref,
                fc1_ref, fc2_ref, ep_ref, kl_ref, sim_s, wm_s, wb_s, bm_s, bb_s):
    @pl.when(pl.program_id(0) == 0)
    def _init():
        kl_ref[...] = jnp.zeros_like(kl_ref)

    xiT = xiT_ref[...]
    xjT = xjT_ref[...]
    w1e = w1e_ref[...]
    be = be_ref[...]
    w2b = w2b_ref[...]
    b2s = b2s_ref[...]
    for l in range(H):
        z = jnp.abs(xiT - xjT[l:l + 1, :])
        y = jnp.dot(w1e, z, preferred_element_type=jnp.float32) + be
        y = jnp.where(y > 0, y, 0.01 * y)
        s5 = jnp.dot(w2b, y, preferred_element_type=jnp.float32) + b2s
        sim_s[l:l + 1, :] = s5[0:1, :]
        wm_s[l:l + 1, :] = s5[1:2, :]
        wb_s[l:l + 1, :] = s5[2:3, :]
        bm_s[l:l + 1, :] = s5[3:4, :]
        bb_s[l:l + 1, :] = s5[4:5, :]
    sv = sim_s[...]
    ep_logits = jnp.dot(fc1_ref[...], sv, preferred_element_type=jnp.float32)
    edge_pred = jax.nn.sigmoid(ep_logits)
    lm = wm_s[...] * sv + bm_s[...]
    lv = jnp.abs(jnp.log(sv * sv * jnp.exp(wb_s[...]) + jnp.exp(bb_s[...])))
    ey_in = jax.nn.sigmoid(lm + lv * nzT_ref[...])
    edge_y = jnp.dot(fc2_ref[...], ey_in, preferred_element_type=jnp.float32)
    logp_x = edge_pred - _logsumexp0(edge_pred)
    logp_y = edge_y - _logsumexp0(edge_y)
    p_y = jnp.exp(logp_y)
    kl_blk = jnp.sum(jnp.sum(p_y * (logp_y - logp_x), axis=1, keepdims=True),
                     axis=0, keepdims=True)
    kl_ref[...] += kl_blk
    ep_ref[...] = jnp.mean(edge_pred, axis=0, keepdims=True)[None]


def _tc_pass2(xiT, xjT, noiseT, W1eff, beff, W2blk, b2s, fc1, fc2):
    return pl.pallas_call(
        _pass2_body,
        grid=(NEB,),
        in_specs=[
            pl.BlockSpec((H, EB), lambda i: (0, i)),
            pl.BlockSpec((H, EB), lambda i: (0, i)),
            pl.BlockSpec((H, EB), lambda i: (0, i)),
            pl.BlockSpec((80, H), lambda i: (0, 0)),
            pl.BlockSpec((80, 1), lambda i: (0, 0)),
            pl.BlockSpec((5, 80), lambda i: (0, 0)),
            pl.BlockSpec((5, 1), lambda i: (0, 0)),
            pl.BlockSpec((4, H), lambda i: (0, 0)),
            pl.BlockSpec((4, H), lambda i: (0, 0)),
        ],
        out_specs=[
            pl.BlockSpec((1, 1, EB), lambda i: (i, 0, 0)),
            pl.BlockSpec((1, 1), lambda i: (0, 0)),
        ],
        out_shape=[
            jax.ShapeDtypeStruct((NEB, 1, EB), jnp.float32),
            jax.ShapeDtypeStruct((1, 1), jnp.float32),
        ],
        scratch_shapes=[pltpu.VMEM((H, EB), jnp.float32) for _ in range(5)],
        interpret=_INTERP,
    )(xiT, xjT, noiseT, W1eff, beff, W2blk, b2s, fc1, fc2)


def _bnstat_body(h1_ref, re_ref, s_ref, q_ref):
    @pl.when(pl.program_id(0) == 0)
    def _init():
        s_ref[...] = jnp.zeros_like(s_ref)
        q_ref[...] = jnp.zeros_like(q_ref)

    x = jnp.concatenate([h1_ref[...], re_ref[...]], axis=1)
    s_ref[...] += jnp.sum(x, axis=0, keepdims=True)
    q_ref[...] += jnp.sum(x * x, axis=0, keepdims=True)


def _tc_bnstat(h1, root_ext):
    return pl.pallas_call(
        _bnstat_body,
        grid=(NRB,),
        in_specs=[
            pl.BlockSpec((RB, H), lambda i: (i, 0)),
            pl.BlockSpec((RB, D_IN), lambda i: (i, 0)),
        ],
        out_specs=[
            pl.BlockSpec((1, H + D_IN), lambda i: (0, 0)),
            pl.BlockSpec((1, H + D_IN), lambda i: (0, 0)),
        ],
        out_shape=[
            jax.ShapeDtypeStruct((1, H + D_IN), jnp.float32),
            jax.ShapeDtypeStruct((1, H + D_IN), jnp.float32),
        ],
        interpret=_INTERP,
    )(h1, root_ext)


def _xw2_body(h1_ref, re_ref, sc_ref, sh_ref, w2_ref, p0_ref, p1_ref,
              xws_ref, dis_ref):
    dis = lax.rsqrt(p0_ref[...] + p1_ref[...] + 1.0)
    x = jnp.concatenate([h1_ref[...], re_ref[...]], axis=1)
    x = jax.nn.relu(x * sc_ref[...] + sh_ref[...])
    xw = jnp.dot(x, w2_ref[...], preferred_element_type=jnp.float32)
    xws_ref[...] = xw * dis[:, 0:1]
    dis_ref[...] = dis


def _tc_xw2(h1, root_ext, scale, shift, W2, p0, p1):
    return pl.pallas_call(
        _xw2_body,
        grid=(NRB,),
        in_specs=[
            pl.BlockSpec((RB, H), lambda i: (i, 0)),
            pl.BlockSpec((RB, D_IN), lambda i: (i, 0)),
            pl.BlockSpec((1, H + D_IN), lambda i: (0, 0)),
            pl.BlockSpec((1, H + D_IN), lambda i: (0, 0)),
            pl.BlockSpec((H + D_IN, D_IN), lambda i: (0, 0)),
            pl.BlockSpec((RB, H), lambda i: (i, 0)),
            pl.BlockSpec((RB, H), lambda i: (i, 0)),
        ],
        out_specs=[
            pl.BlockSpec((RB, D_IN), lambda i: (i, 0)),
            pl.BlockSpec((RB, H), lambda i: (i, 0)),
        ],
        out_shape=[
            jax.ShapeDtypeStruct((N, D_IN), jnp.float32),
            jax.ShapeDtypeStruct((N, H), jnp.float32),
        ],
        interpret=_INTERP,
    )(h1, root_ext, scale, shift, W2, p0, p1)


def _out_body(p0a_ref, p1a_ref, p0b_ref, p1b_ref, xws_ref, dis_ref, b2_ref,
              re2_ref, bat_ref, out_ref, sums, cnts):
    @pl.when(pl.program_id(0) == 0)
    def _init():
        sums[...] = jnp.zeros_like(sums)
        cnts[...] = jnp.zeros_like(cnts)

    psum = jnp.concatenate([p0a_ref[...] + p1a_ref[...],
                            p0b_ref[...] + p1b_ref[...]], axis=1)
    x2 = jax.nn.relu(dis_ref[...][:, 0:1]
                     * (psum + xws_ref[...]) + b2_ref[...])
    z = jnp.concatenate([x2, re2_ref[...]], axis=1)
    bat = bat_ref[0]
    oh = (lax.broadcasted_iota(jnp.int32, (G, RB), 0) == bat).astype(jnp.float32)
    sums[...] += jnp.dot(oh, z, preferred_element_type=jnp.float32)
    cnts[...] += jnp.sum(oh, axis=1, keepdims=True)

    @pl.when(pl.program_id(0) == NRB - 1)
    def _fin():
        out_ref[...] = sums[...] / jnp.maximum(cnts[...], 1.0)


def _tc_out(p0a, p1a, p0b, p1b, xws2, dis2, b2, root_ext2, batch3):
    HD = D_IN // 2
    return pl.pallas_call(
        _out_body,
        grid=(NRB,),
        in_specs=[
            pl.BlockSpec((RB, HD), lambda i: (i, 0)),
            pl.BlockSpec((RB, HD), lambda i: (i, 0)),
            pl.BlockSpec((RB, HD), lambda i: (i, 0)),
            pl.BlockSpec((RB, HD), lambda i: (i, 0)),
            pl.BlockSpec((RB, D_IN), lambda i: (i, 0)),
            pl.BlockSpec((RB, H), lambda i: (i, 0)),
            pl.BlockSpec((1, D_IN), lambda i: (0, 0)),
            pl.BlockSpec((RB, H), lambda i: (i, 0)),
            pl.BlockSpec((1, 1, RB), lambda i: (i, 0, 0)),
        ],
        out_specs=pl.BlockSpec((G, H + D_IN), lambda i: (0, 0)),
        out_shape=jax.ShapeDtypeStruct((G, H + D_IN), jnp.float32),
        scratch_shapes=[
            pltpu.VMEM((G, H + D_IN), jnp.float32),
            pltpu.VMEM((G, 1), jnp.float32),
        ],
        interpret=_INTERP,
    )(p0a, p1a, p0b, p1b, xws2, dis2, b2.reshape(1, D_IN), root_ext2, batch3)


from jax.experimental.pallas import tpu_sc as plsc

NC = 2
NS = 16
NW = NC * NS
CH = 128
E_PAD = 163840
EPT = E_PAD // NW
NCHUNK = EPT // CH
N_PAD = 10240
ZPT = N_PAD // NS
NPT = N_PAD // NW
RCH = 64


def _sc_mesh():
    return plsc.VectorSubcoreMesh(core_axis_name="c", subcore_axis_name="s")


_SC_PARAMS = dict(compiler_params=pltpu.CompilerParams(use_tc_tiling_on_sc=False))


def _zero_vmem_rows(ref, nrows, width):
    def zrow(r, _):
        for j in range(width // 16):
            ref[r, pl.ds(16 * j, 16)] = jnp.zeros((16,), jnp.float32)
        return 0
    lax.fori_loop(0, nrows, zrow, 0)


def _sc_hist(col2d, wrows_pad=None):
    has_w = wrows_pad is not None

    @functools.partial(
        pl.kernel, mesh=_sc_mesh(), **_SC_PARAMS,
        out_type=[
            jax.ShapeDtypeStruct((N_PAD, H), jnp.float32),
            jax.ShapeDtypeStruct((N_PAD, H), jnp.float32),
        ],
        scratch_types=[
            pltpu.VMEM((NCHUNK, CH), jnp.int32),
            pltpu.VMEM((CH, H), jnp.float32),
            pltpu.VMEM((CH, H), jnp.float32),
            pltpu.VMEM((ZPT, H), jnp.float32),
            pltpu.VMEM_SHARED((N_PAD, H), jnp.float32),
            pltpu.SemaphoreType.DMA,
            pltpu.SemaphoreType.DMA,
        ],
    )
    def k(col_hbm, w_hbm, out0, out1, cidx_v, rows0_v, rows1_v, ztile_v,
          accum, sem0, sem1):
        cid = lax.axis_index("c")
        sid = lax.axis_index("s")
        wid = cid * NS + sid
        _zero_vmem_rows(ztile_v, ZPT, H)
        pltpu.sync_copy(ztile_v, accum.at[pl.ds(sid * ZPT, ZPT)])
        pltpu.sync_copy(col_hbm.at[pl.ds(wid * NCHUNK, NCHUNK)], cidx_v)
        if not has_w:
            def onerow(r, _):
                rows0_v[r, :] = jnp.ones((16,), jnp.float32)
                return 0
            lax.fori_loop(0, CH, onerow, 0)
        plsc.subcore_barrier()

        if has_w:
            bufs = (rows0_v, rows1_v)
            sems = (sem0, sem1)

            def start(i, b):
                pltpu.async_copy(
                    w_hbm.at[pl.ds(wid * EPT + i * CH, CH)], bufs[b], sems[b])

            start(0, 0)

            def pair(it, _):
                pltpu.async_copy(
                    w_hbm.at[pl.ds(wid * EPT + (2 * it + 1) * CH, CH)],
                    rows1_v, sem1)
                pltpu.make_async_copy(
                    w_hbm.at[pl.ds(0, CH)], rows0_v, sem0).wait()
                pltpu.sync_copy(rows0_v, accum.at[cidx_v.at[2 * it]], add=True)

                @pl.when(it < NCHUNK // 2 - 1)
                def _pre():
                    pltpu.async_copy(
                        w_hbm.at[pl.ds(wid * EPT + (2 * it + 2) * CH, CH)],
                        rows0_v, sem0)
                pltpu.make_async_copy(
                    w_hbm.at[pl.ds(0, CH)], rows1_v, sem1).wait()
                pltpu.sync_copy(rows1_v, accum.at[cidx_v.at[2 * it + 1]],
                                add=True)
                return 0
            lax.fori_loop(0, NCHUNK // 2, pair, 0)
        else:
            def chunk(i, _):
                pltpu.sync_copy(rows0_v, accum.at[cidx_v.at[i]], add=True)
                return 0
            lax.fori_loop(0, NCHUNK, chunk, 0)
        plsc.subcore_barrier()

        @pl.when(cid == 0)
        def _c0():
            pltpu.sync_copy(accum.at[pl.ds(sid * ZPT, ZPT)],
                            out0.at[pl.ds(sid * ZPT, ZPT)])

        @pl.when(cid == 1)
        def _c1():
            pltpu.sync_copy(accum.at[pl.ds(sid * ZPT, ZPT)],
                            out1.at[pl.ds(sid * ZPT, ZPT)])

    if not has_w:
        wrows_pad = jnp.zeros((8, H), jnp.float32)
    return k(col2d, wrows_pad)


def _sc_conv_scatter(table, row_pad, col_pad, D, ewrows=None):
    has_w = ewrows is not None

    @functools.partial(
        pl.kernel, mesh=_sc_mesh(), **_SC_PARAMS,
        out_type=[
            jax.ShapeDtypeStruct((N_PAD, D), jnp.float32),
            jax.ShapeDtypeStruct((N_PAD, D), jnp.float32),
        ],
        scratch_types=[
            pltpu.VMEM((NCHUNK, CH), jnp.int32),
            pltpu.VMEM((NCHUNK, CH), jnp.int32),
            pltpu.VMEM((CH, H), jnp.float32),
            pltpu.VMEM((CH, H), jnp.float32),
            pltpu.VMEM((CH, D), jnp.float32),
            pltpu.VMEM((CH, D), jnp.float32),
            pltpu.VMEM((ZPT, D), jnp.float32),
            pltpu.VMEM_SHARED((N_PAD, D), jnp.float32),
            pltpu.SemaphoreType.DMA,
            pltpu.SemaphoreType.DMA,
            pltpu.SemaphoreType.DMA,
            pltpu.SemaphoreType.DMA,
        ],
    )
    def k(table_hbm, row_hbm, col_hbm, w_hbm, out0, out1,
          ridx_v, cidx_v, w0_v, w1_v, rows0_v, rows1_v, ztile_v,
          accum, sem0, sem1, wsem0, wsem1):
        cid = lax.axis_index("c")
        sid = lax.axis_index("s")
        wid = cid * NS + sid
        _zero_vmem_rows(ztile_v, ZPT, D)
        pltpu.sync_copy(ztile_v, accum.at[pl.ds(sid * ZPT, ZPT)])
        pltpu.sync_copy(row_hbm.at[pl.ds(wid * NCHUNK, NCHUNK)], ridx_v)
        pltpu.sync_copy(col_hbm.at[pl.ds(wid * NCHUNK, NCHUNK)], cidx_v)
        plsc.subcore_barrier()

        rbufs = (rows0_v, rows1_v)
        sems = (sem0, sem1)
        wbufs = (w0_v, w1_v)
        wsems = (wsem0, wsem1)

        def start(i, b):
            pltpu.async_copy(table_hbm.at[ridx_v.at[i]], rbufs[b], sems[b])
            if has_w:
                pltpu.async_copy(
                    w_hbm.at[pl.ds(wid * EPT + i * CH, CH)], wbufs[b],
                    wsems[b])

        def finish(i, b):
            pltpu.make_async_copy(
                table_hbm.at[pl.ds(0, CH)], rbufs[b], sems[b]).wait()
            if has_w:
                pltpu.make_async_copy(
                    w_hbm.at[pl.ds(0, CH)], wbufs[b], wsems[b]).wait()

                def scale(e, _):
                    bc = wbufs[b][e, :]
                    for j in range(D // 16):
                        s = pl.ds(16 * j, 16)
                        rbufs[b][e, s] = rbufs[b][e, s] * bc
                    return 0
                lax.fori_loop(0, CH, scale, 0)
            pltpu.sync_copy(rbufs[b], accum.at[cidx_v.at[i]], add=True)

        start(0, 0)

        def pair(it, _):
            start(2 * it + 1, 1)
            finish(2 * it, 0)

            @pl.when(it < NCHUNK // 2 - 1)
            def _pre():
                start(2 * it + 2, 0)
            finish(2 * it + 1, 1)
            return 0
        lax.fori_loop(0, NCHUNK // 2, pair, 0)
        plsc.subcore_barrier()

        @pl.when(cid == 0)
        def _c0():
            pltpu.sync_copy(accum.at[pl.ds(sid * ZPT, ZPT)],
                            out0.at[pl.ds(sid * ZPT, ZPT)])

        @pl.when(cid == 1)
        def _c1():
            pltpu.sync_copy(accum.at[pl.ds(sid * ZPT, ZPT)],
                            out1.at[pl.ds(sid * ZPT, ZPT)])

    if not has_w:
        ewrows = jnp.zeros((E_PAD, H), jnp.float32)
    return k(table, row_pad, col_pad, ewrows)


def _sc_edge_gather(h1, row_pad, col_pad):
    @functools.partial(
        pl.kernel, mesh=_sc_mesh(), **_SC_PARAMS,
        out_type=[
            jax.ShapeDtypeStruct((E_PAD, H), jnp.float32),
            jax.ShapeDtypeStruct((E_PAD, H), jnp.float32),
        ],
        scratch_types=[
            pltpu.VMEM((2 * NCHUNK, CH), jnp.int32),
            pltpu.VMEM((CH, H), jnp.float32),
            pltpu.VMEM((CH, H), jnp.float32),
            pltpu.SemaphoreType.DMA,
            pltpu.SemaphoreType.DMA,
        ],
    )
    def k(h1_hbm, row_hbm, col_hbm, xi_hbm, xj_hbm, idx_v, rows0_v, rows1_v,
          sem0, sem1):
        cid = lax.axis_index("c")
        sid = lax.axis_index("s")
        wid = cid * NS + sid
        pltpu.sync_copy(row_hbm.at[pl.ds(wid * NCHUNK, NCHUNK)],
                        idx_v.at[pl.ds(0, NCHUNK)])
        pltpu.sync_copy(col_hbm.at[pl.ds(wid * NCHUNK, NCHUNK)],
                        idx_v.at[pl.ds(NCHUNK, NCHUNK)])

        def shift(r, _):
            for t in range(CH // 16):
                s = pl.ds(16 * t, 16)
                v = idx_v[r, s] - 1
                idx_v[r, s] = jnp.where(v < 0, v + N, v)
            return 0
        lax.fori_loop(0, 2 * NCHUNK, shift, 0)

        rbufs = (rows0_v, rows1_v)
        sems = (sem0, sem1)

        def start(i, b):
            pltpu.async_copy(h1_hbm.at[idx_v.at[i]], rbufs[b], sems[b])

        def finish(i, b):
            pltpu.make_async_copy(
                h1_hbm.at[pl.ds(0, CH)], rbufs[b], sems[b]).wait()
            half = i // NCHUNK
            j = i - half * NCHUNK
            base = wid * EPT + j * CH

            @pl.when(half == 0)
            def _xi():
                pltpu.sync_copy(rbufs[b], xi_hbm.at[pl.ds(base, CH)])

            @pl.when(half == 1)
            def _xj():
                pltpu.sync_copy(rbufs[b], xj_hbm.at[pl.ds(base, CH)])

        start(0, 0)

        def pair(it, _):
            start(2 * it + 1, 1)
            finish(2 * it, 0)

            @pl.when(it < NCHUNK - 1)
            def _pre():
                start(2 * it + 2, 0)
            finish(2 * it + 1, 1)
            return 0
        lax.fori_loop(0, NCHUNK, pair, 0)

    return k(h1, row_pad, col_pad)


def _sc_root_gather(x0, h1, root_index, batch_pad):
    @functools.partial(
        pl.kernel, mesh=_sc_mesh(), **_SC_PARAMS,
        out_type=[
            jax.ShapeDtypeStruct((N_PAD, D_IN), jnp.float32),
            jax.ShapeDtypeStruct((N_PAD, H), jnp.float32),
        ],
        scratch_types=[
            pltpu.VMEM((RCH,), jnp.int32),
            pltpu.VMEM((RCH,), jnp.int32),
            pltpu.VMEM((RCH, D_IN), jnp.float32),
            pltpu.VMEM((RCH, H), jnp.float32),
            pltpu.SemaphoreType.DMA,
        ],
    )
    def k(x0_hbm, h1_hbm, ri_hbm, bat_hbm, re_hbm, re2_hbm,
          braw_v, idx_v, rows1_v, rows2_v, sem):
        cid = lax.axis_index("c")
        sid = lax.axis_index("s")
        wid = cid * NS + sid

        def chunk(i, _):
            base = wid * NPT + i * RCH
            pltpu.sync_copy(bat_hbm.at[pl.ds(base, RCH)], braw_v)
            pltpu.async_copy(ri_hbm.at[braw_v], idx_v, sem).wait()
            pltpu.async_copy(x0_hbm.at[idx_v], rows1_v, sem).wait()
            pltpu.sync_copy(rows1_v, re_hbm.at[pl.ds(base, RCH)])
            pltpu.async_copy(h1_hbm.at[idx_v], rows2_v, sem).wait()
            pltpu.sync_copy(rows2_v, re2_hbm.at[pl.ds(base, RCH)])
            return 0
        lax.fori_loop(0, NPT // RCH, chunk, 0)

    return k(x0, h1, root_index, batch_pad)


def kernel(node_features, edge_index, root_index, batch_size, params, noise):
    x0 = node_features
    row, col = edge_index[0], edge_index[1]

    npad = E_PAD - E
    pad_src = (jnp.arange(npad, dtype=jnp.int32) % 240)
    pad_dst = N + (jnp.arange(npad, dtype=jnp.int32) % (N_PAD - N))
    row2d = jnp.concatenate([row, pad_src]).reshape(E_PAD // CH, CH)
    col2d = jnp.concatenate([col, pad_dst]).reshape(E_PAD // CH, CH)
    batch_pad = jnp.concatenate(
        [batch_size, jnp.zeros((N_PAD - N,), jnp.int32)])

    d1p0, d1p1 = _sc_hist(col2d)
    xws1, dis1 = _tc_xw1(x0, params['W1'], d1p0, d1p1)

    c1p0, c1p1 = _sc_conv_scatter(xws1, row2d, col2d, H)
    h1 = _tc_h1(c1p0, c1p1, xws1, dis1, params['b1'])

    xi, xj = _sc_edge_gather(h1, row2d, col2d)
    root_ext, root_ext2 = _sc_root_gather(x0, h1, root_index, batch_pad)

    xiT = xi[:E].T
    xjT = xj[:E].T
    noiseT = noise[:, 0, :].T

    Ssum, Msum = _tc_moments(xiT, xjT)
    cntm = float(E * H)
    Mm = Msum[:, 0] / cntm
    Sm = Ssum / cntm
    w1e_rows, be_rows, w2b_rows, b2s_rows = [], [], [], []
    for k, name in enumerate(('sim', 'wm', 'wb', 'bm', 'bb')):
        p = params[name]
        w1 = p['w1']
        mean = w1 @ Mm
        ey2 = jnp.einsum('oc,cd,od->o', w1, Sm, w1)
        var = ey2 - mean * mean
        sc = p['g'] * lax.rsqrt(var + EPS)
        w1e_rows.append(w1 * sc[:, None])
        be_rows.append(p['b'] - mean * sc)
        wrow = jnp.zeros((5, 80), jnp.float32).at[k, 16 * k:16 * k + 16].set(p['w2'][0])
        w2b_rows.append(wrow)
        b2s_rows.append(p['b2'][0])
    W1eff = jnp.concatenate(w1e_rows, axis=0)
    beff = jnp.concatenate(be_rows).reshape(80, 1)
    W2blk = sum(w2b_rows)
    b2s = jnp.stack(b2s_rows).reshape(5, 1)

    ep3, klsum = _tc_pass2(xiT, xjT, noiseT, W1eff, beff, W2blk, b2s,
                           params['fc1'], params['fc2'])
    ep = ep3.reshape(E)
    edge_loss = klsum[0, 0] / float(E)

    s1, q1 = _tc_bnstat(h1, root_ext)
    mb = s1 / float(N)
    vb = q1 / float(N) - mb * mb
    scb = params['bn1_g'].reshape(1, -1) * lax.rsqrt(vb + EPS)
    shb = params['bn1_b'].reshape(1, -1) - mb * scb

    ep_rows = jnp.concatenate(
        [jnp.broadcast_to(ep[:, None], (E, H)),
         jnp.zeros((E_PAD - E, H), jnp.float32)])
    d2p0, d2p1 = _sc_hist(col2d, ep_rows)
    xws2, dis2 = _tc_xw2(h1, root_ext, scb, shb, params['W2'], d2p0, d2p1)
    HD = D_IN // 2
    c2p0a, c2p1a = _sc_conv_scatter(xws2[:, :HD], row2d, col2d, HD,
                                    ewrows=ep_rows)
    c2p0b, c2p1b = _sc_conv_scatter(xws2[:, HD:], row2d, col2d, HD,
                                    ewrows=ep_rows)

    batch3 = batch_size.reshape(NRB, 1, RB)
    out = _tc_out(c2p0a, c2p1a, c2p0b, c2p1b, xws2, dis2, params['b2'],
                  root_ext2, batch3)
    return out, edge_loss


# speedup vs baseline: 9.9271x; 1.3818x over previous
import functools

import jax
import jax.numpy as jnp
import numpy as np
from jax import lax
from jax.experimental import pallas as pl
from jax.experimental.pallas import tpu as pltpu

N = 10000
E = 160000
D_IN = 128
H = 16
G = 64
EPS = 1e-5

_INTERP = False

EB = 640
NEB = E // EB
RB = 1000
NRB = N // RB


def _xw1_body(x0_ref, w1_ref, p0_ref, p1_ref, xws_ref, dis_ref):
    dis = lax.rsqrt(p0_ref[...] + p1_ref[...] + 1.0)
    xw = jnp.dot(x0_ref[...], w1_ref[...], preferred_element_type=jnp.float32)
    xws_ref[...] = xw * dis
    dis_ref[...] = dis


def _tc_xw1(x0, W1, p0, p1):
    return pl.pallas_call(
        _xw1_body,
        grid=(NRB,),
        in_specs=[
            pl.BlockSpec((RB, D_IN), lambda i: (i, 0)),
            pl.BlockSpec((D_IN, H), lambda i: (0, 0)),
            pl.BlockSpec((RB, H), lambda i: (i, 0)),
            pl.BlockSpec((RB, H), lambda i: (i, 0)),
        ],
        out_specs=[
            pl.BlockSpec((RB, H), lambda i: (i, 0)),
            pl.BlockSpec((RB, H), lambda i: (i, 0)),
        ],
        out_shape=[
            jax.ShapeDtypeStruct((N, H), jnp.float32),
            jax.ShapeDtypeStruct((N, H), jnp.float32),
        ],
        interpret=_INTERP,
    )(x0, W1, p0, p1)


def _h1_body(p0_ref, p1_ref, xws_ref, dis_ref, b_ref, o_ref):
    o_ref[...] = dis_ref[...] * (p0_ref[...] + p1_ref[...] + xws_ref[...]) + b_ref[...]


def _tc_h1(p0, p1, xws1, dis1, b1):
    return pl.pallas_call(
        _h1_body,
        grid=(NRB,),
        in_specs=[
            pl.BlockSpec((RB, H), lambda i: (i, 0)),
            pl.BlockSpec((RB, H), lambda i: (i, 0)),
            pl.BlockSpec((RB, H), lambda i: (i, 0)),
            pl.BlockSpec((RB, H), lambda i: (i, 0)),
            pl.BlockSpec((1, H), lambda i: (0, 0)),
        ],
        out_specs=pl.BlockSpec((RB, H), lambda i: (i, 0)),
        out_shape=jax.ShapeDtypeStruct((N, H), jnp.float32),
        interpret=_INTERP,
    )(p0, p1, xws1, dis1, b1.reshape(1, H))


def _build_z17(zb_s, xiT, xjT):
    for l in range(H):
        zb_s[0:H, l * EB:(l + 1) * EB] = jnp.abs(xiT - xjT[l:l + 1, :])


def _moments_body(xiT_ref, xjT_ref, s_ref, zb_s):
    @pl.when(pl.program_id(0) == 0)
    def _init():
        s_ref[...] = jnp.zeros_like(s_ref)
        zb_s[H:H + 1, :] = jnp.ones((1, H * EB), jnp.float32)

    _build_z17(zb_s, xiT_ref[...], xjT_ref[...])
    zb = zb_s[...]
    s_ref[...] += lax.dot_general(zb, zb, (((1,), (1,)), ((), ())),
                                  preferred_element_type=jnp.float32)


def _tc_moments(xiT, xjT):
    return pl.pallas_call(
        _moments_body,
        grid=(NEB,),
        in_specs=[
            pl.BlockSpec((H, EB), lambda i: (0, i)),
            pl.BlockSpec((H, EB), lambda i: (0, i)),
        ],
        out_specs=pl.BlockSpec((H + 1, H + 1), lambda i: (0, 0)),
        out_shape=jax.ShapeDtypeStruct((H + 1, H + 1), jnp.float32),
        scratch_shapes=[pltpu.VMEM((H + 1, H * EB), jnp.float32)],
        interpret=_INTERP,
    )(xiT, xjT)


def _logsumexp0(x):
    mx = jnp.max(x, axis=0, keepdims=True)
    return jnp.log(jnp.sum(jnp.exp(x - mx), axis=0, keepdims=True)) + mx


def _pass2_body(xiT_ref, xjT_ref, nzT_ref, w1e_ref, w2b_ref, b2s_ref,
                fc1_ref, fc2_ref, ep_ref, kl_ref, zb_s,
                sim_s, wm_s, wb_s, bm_s, bb_s):
    @pl.when(pl.program_id(0) == 0)
    def _init():
        kl_ref[...] = jnp.zeros_like(kl_ref)

    @pl.when(pl.program_id(0) == 0)
    def _ones():
        zb_s[H:H + 1, :] = jnp.ones((1, H * EB), jnp.float32)

    _build_z17(zb_s, xiT_ref[...], xjT_ref[...])
    w2b = w2b_ref[...]
    b2s = b2s_ref[...]
    y = jnp.dot(w1e_ref[...], zb_s[...],
                preferred_element_type=jnp.float32)
    y = jnp.where(y > 0, y, 0.01 * y)
    for l in range(H):
        s5 = jnp.dot(w2b, y[:, l * EB:(l + 1) * EB],
                     preferred_element_type=jnp.float32) + b2s
        sim_s[l:l + 1, :] = s5[0:1, :]
        wm_s[l:l + 1, :] = s5[1:2, :]
        wb_s[l:l + 1, :] = s5[2:3, :]
        bm_s[l:l + 1, :] = s5[3:4, :]
        bb_s[l:l + 1, :] = s5[4:5, :]
    sv = sim_s[...]
    ep_logits = jnp.dot(fc1_ref[...], sv, preferred_element_type=jnp.float32)
    edge_pred = jax.nn.sigmoid(ep_logits)
    lm = wm_s[...] * sv + bm_s[...]
    lv = jnp.abs(jnp.log(sv * sv * jnp.exp(wb_s[...]) + jnp.exp(bb_s[...])))
    ey_in = jax.nn.sigmoid(lm + lv * nzT_ref[...])
    edge_y = jnp.dot(fc2_ref[...], ey_in, preferred_element_type=jnp.float32)
    logp_x = edge_pred - _logsumexp0(edge_pred)
    logp_y = edge_y - _logsumexp0(edge_y)
    p_y = jnp.exp(logp_y)
    kl_blk = jnp.sum(jnp.sum(p_y * (logp_y - logp_x), axis=1, keepdims=True),
                     axis=0, keepdims=True)
    kl_ref[...] += kl_blk
    ep_ref[...] = jnp.mean(edge_pred, axis=0, keepdims=True)[None]


def _tc_pass2(xiT, xjT, noiseT, W1eff17, W2blk, b2s, fc1, fc2):
    return pl.pallas_call(
        _pass2_body,
        grid=(NEB,),
        in_specs=[
            pl.BlockSpec((H, EB), lambda i: (0, i)),
            pl.BlockSpec((H, EB), lambda i: (0, i)),
            pl.BlockSpec((H, EB), lambda i: (0, i)),
            pl.BlockSpec((80, H + 1), lambda i: (0, 0)),
            pl.BlockSpec((5, 80), lambda i: (0, 0)),
            pl.BlockSpec((5, 1), lambda i: (0, 0)),
            pl.BlockSpec((4, H), lambda i: (0, 0)),
            pl.BlockSpec((4, H), lambda i: (0, 0)),
        ],
        out_specs=[
            pl.BlockSpec((1, 1, EB), lambda i: (i, 0, 0)),
            pl.BlockSpec((1, 1), lambda i: (0, 0)),
        ],
        out_shape=[
            jax.ShapeDtypeStruct((NEB, 1, EB), jnp.float32),
            jax.ShapeDtypeStruct((1, 1), jnp.float32),
        ],
        scratch_shapes=[pltpu.VMEM((H + 1, H * EB), jnp.float32)]
        + [pltpu.VMEM((H, EB), jnp.float32) for _ in range(5)],
        interpret=_INTERP,
    )(xiT, xjT, noiseT, W1eff17, W2blk, b2s, fc1, fc2)


def _bnstat_body(h1_ref, re_ref, s_ref, q_ref):
    @pl.when(pl.program_id(0) == 0)
    def _init():
        s_ref[...] = jnp.zeros_like(s_ref)
        q_ref[...] = jnp.zeros_like(q_ref)

    x = jnp.concatenate([h1_ref[...], re_ref[...]], axis=1)
    s_ref[...] += jnp.sum(x, axis=0, keepdims=True)
    q_ref[...] += jnp.sum(x * x, axis=0, keepdims=True)


def _tc_bnstat(h1, root_ext):
    return pl.pallas_call(
        _bnstat_body,
        grid=(NRB,),
        in_specs=[
            pl.BlockSpec((RB, H), lambda i: (i, 0)),
            pl.BlockSpec((RB, D_IN), lambda i: (i, 0)),
        ],
        out_specs=[
            pl.BlockSpec((1, H + D_IN), lambda i: (0, 0)),
            pl.BlockSpec((1, H + D_IN), lambda i: (0, 0)),
        ],
        out_shape=[
            jax.ShapeDtypeStruct((1, H + D_IN), jnp.float32),
            jax.ShapeDtypeStruct((1, H + D_IN), jnp.float32),
        ],
        interpret=_INTERP,
    )(h1, root_ext)


def _xw2_body(h1_ref, re_ref, sc_ref, sh_ref, w2_ref, p0_ref, p1_ref,
              xws_ref, dis_ref):
    dis = lax.rsqrt(p0_ref[...] + p1_ref[...] + 1.0)
    x = jnp.concatenate([h1_ref[...], re_ref[...]], axis=1)
    x = jax.nn.relu(x * sc_ref[...] + sh_ref[...])
    xw = jnp.dot(x, w2_ref[...], preferred_element_type=jnp.float32)
    xws_ref[...] = xw * dis[:, 0:1]
    dis_ref[...] = dis


def _tc_xw2(h1, root_ext, scale, shift, W2, p0, p1):
    return pl.pallas_call(
        _xw2_body,
        grid=(NRB,),
        in_specs=[
            pl.BlockSpec((RB, H), lambda i: (i, 0)),
            pl.BlockSpec((RB, D_IN), lambda i: (i, 0)),
            pl.BlockSpec((1, H + D_IN), lambda i: (0, 0)),
            pl.BlockSpec((1, H + D_IN), lambda i: (0, 0)),
            pl.BlockSpec((H + D_IN, D_IN), lambda i: (0, 0)),
            pl.BlockSpec((RB, H), lambda i: (i, 0)),
            pl.BlockSpec((RB, H), lambda i: (i, 0)),
        ],
        out_specs=[
            pl.BlockSpec((RB, D_IN), lambda i: (i, 0)),
            pl.BlockSpec((RB, H), lambda i: (i, 0)),
        ],
        out_shape=[
            jax.ShapeDtypeStruct((N, D_IN), jnp.float32),
            jax.ShapeDtypeStruct((N, H), jnp.float32),
        ],
        interpret=_INTERP,
    )(h1, root_ext, scale, shift, W2, p0, p1)


def _out_body(p0a_ref, p1a_ref, p0b_ref, p1b_ref, xws_ref, dis_ref, b2_ref,
              re2_ref, bat_ref, out_ref, sums, cnts):
    @pl.when(pl.program_id(0) == 0)
    def _init():
        sums[...] = jnp.zeros_like(sums)
        cnts[...] = jnp.zeros_like(cnts)

    psum = jnp.concatenate([p0a_ref[...] + p1a_ref[...],
                            p0b_ref[...] + p1b_ref[...]], axis=1)
    x2 = jax.nn.relu(dis_ref[...][:, 0:1]
                     * (psum + xws_ref[...]) + b2_ref[...])
    z = jnp.concatenate([x2, re2_ref[...]], axis=1)
    bat = bat_ref[0]
    oh = (lax.broadcasted_iota(jnp.int32, (G, RB), 0) == bat).astype(jnp.float32)
    sums[...] += jnp.dot(oh, z, preferred_element_type=jnp.float32)
    cnts[...] += jnp.sum(oh, axis=1, keepdims=True)

    @pl.when(pl.program_id(0) == NRB - 1)
    def _fin():
        out_ref[...] = sums[...] / jnp.maximum(cnts[...], 1.0)


def _tc_out(p0a, p1a, p0b, p1b, xws2, dis2, b2, root_ext2, batch3):
    HD = D_IN // 2
    return pl.pallas_call(
        _out_body,
        grid=(NRB,),
        in_specs=[
            pl.BlockSpec((RB, HD), lambda i: (i, 0)),
            pl.BlockSpec((RB, HD), lambda i: (i, 0)),
            pl.BlockSpec((RB, HD), lambda i: (i, 0)),
            pl.BlockSpec((RB, HD), lambda i: (i, 0)),
            pl.BlockSpec((RB, D_IN), lambda i: (i, 0)),
            pl.BlockSpec((RB, H), lambda i: (i, 0)),
            pl.BlockSpec((1, D_IN), lambda i: (0, 0)),
            pl.BlockSpec((RB, H), lambda i: (i, 0)),
            pl.BlockSpec((1, 1, RB), lambda i: (i, 0, 0)),
        ],
        out_specs=pl.BlockSpec((G, H + D_IN), lambda i: (0, 0)),
        out_shape=jax.ShapeDtypeStruct((G, H + D_IN), jnp.float32),
        scratch_shapes=[
            pltpu.VMEM((G, H + D_IN), jnp.float32),
            pltpu.VMEM((G, 1), jnp.float32),
        ],
        interpret=_INTERP,
    )(p0a, p1a, p0b, p1b, xws2, dis2, b2.reshape(1, D_IN), root_ext2, batch3)


from jax.experimental.pallas import tpu_sc as plsc

NC = 2
NS = 16
NW = NC * NS
CH = 128
E_PAD = 163840
EPT = E_PAD // NW
NCHUNK = EPT // CH
N_PAD = 10240
ZPT = N_PAD // NS
NPT = N_PAD // NW
RCH = 64


def _sc_mesh():
    return plsc.VectorSubcoreMesh(core_axis_name="c", subcore_axis_name="s")


_SC_PARAMS = dict(compiler_params=pltpu.CompilerParams(use_tc_tiling_on_sc=False))


def _zero_vmem_rows(ref, nrows, width):
    def zrow(r, _):
        for j in range(width // 16):
            ref[r, pl.ds(16 * j, 16)] = jnp.zeros((16,), jnp.float32)
        return 0
    lax.fori_loop(0, nrows, zrow, 0)


def _sc_hist(col2d, wrows_pad=None):
    has_w = wrows_pad is not None

    @functools.partial(
        pl.kernel, mesh=_sc_mesh(), **_SC_PARAMS,
        out_type=[
            jax.ShapeDtypeStruct((N_PAD, H), jnp.float32),
            jax.ShapeDtypeStruct((N_PAD, H), jnp.float32),
        ],
        scratch_types=[
            pltpu.VMEM((NCHUNK, CH), jnp.int32),
            pltpu.VMEM((CH, H), jnp.float32),
            pltpu.VMEM((CH, H), jnp.float32),
            pltpu.VMEM((ZPT, H), jnp.float32),
            pltpu.VMEM_SHARED((N_PAD, H), jnp.float32),
            pltpu.SemaphoreType.DMA,
            pltpu.SemaphoreType.DMA,
        ],
    )
    def k(col_hbm, w_hbm, out0, out1, cidx_v, rows0_v, rows1_v, ztile_v,
          accum, sem0, sem1):
        cid = lax.axis_index("c")
        sid = lax.axis_index("s")
        wid = cid * NS + sid
        _zero_vmem_rows(ztile_v, ZPT, H)
        pltpu.sync_copy(ztile_v, accum.at[pl.ds(sid * ZPT, ZPT)])
        pltpu.sync_copy(col_hbm.at[pl.ds(wid * NCHUNK, NCHUNK)], cidx_v)
        if not has_w:
            def onerow(r, _):
                rows0_v[r, :] = jnp.ones((16,), jnp.float32)
                return 0
            lax.fori_loop(0, CH, onerow, 0)
        plsc.subcore_barrier()

        if has_w:
            bufs = (rows0_v, rows1_v)
            sems = (sem0, sem1)

            def start(i, b):
                pltpu.async_copy(
                    w_hbm.at[pl.ds(wid * EPT + i * CH, CH)], bufs[b], sems[b])

            start(0, 0)

            def pair(it, _):
                pltpu.async_copy(
                    w_hbm.at[pl.ds(wid * EPT + (2 * it + 1) * CH, CH)],
                    rows1_v, sem1)
                pltpu.make_async_copy(
                    w_hbm.at[pl.ds(0, CH)], rows0_v, sem0).wait()
                pltpu.sync_copy(rows0_v, accum.at[cidx_v.at[2 * it]], add=True)

                @pl.when(it < NCHUNK // 2 - 1)
                def _pre():
                    pltpu.async_copy(
                        w_hbm.at[pl.ds(wid * EPT + (2 * it + 2) * CH, CH)],
                        rows0_v, sem0)
                pltpu.make_async_copy(
                    w_hbm.at[pl.ds(0, CH)], rows1_v, sem1).wait()
                pltpu.sync_copy(rows1_v, accum.at[cidx_v.at[2 * it + 1]],
                                add=True)
                return 0
            lax.fori_loop(0, NCHUNK // 2, pair, 0)
        else:
            def chunk(i, _):
                pltpu.sync_copy(rows0_v, accum.at[cidx_v.at[i]], add=True)
                return 0
            lax.fori_loop(0, NCHUNK, chunk, 0)
        plsc.subcore_barrier()

        @pl.when(cid == 0)
        def _c0():
            pltpu.sync_copy(accum.at[pl.ds(sid * ZPT, ZPT)],
                            out0.at[pl.ds(sid * ZPT, ZPT)])

        @pl.when(cid == 1)
        def _c1():
            pltpu.sync_copy(accum.at[pl.ds(sid * ZPT, ZPT)],
                            out1.at[pl.ds(sid * ZPT, ZPT)])

    if not has_w:
        wrows_pad = jnp.zeros((8, H), jnp.float32)
    return k(col2d, wrows_pad)


def _sc_conv_scatter(table, row_pad, col_pad, D, ewrows=None):
    has_w = ewrows is not None

    @functools.partial(
        pl.kernel, mesh=_sc_mesh(), **_SC_PARAMS,
        out_type=[
            jax.ShapeDtypeStruct((N_PAD, D), jnp.float32),
            jax.ShapeDtypeStruct((N_PAD, D), jnp.float32),
        ],
        scratch_types=[
            pltpu.VMEM((NCHUNK, CH), jnp.int32),
            pltpu.VMEM((NCHUNK, CH), jnp.int32),
            pltpu.VMEM((CH, H), jnp.float32),
            pltpu.VMEM((CH, H), jnp.float32),
            pltpu.VMEM((CH, D), jnp.float32),
            pltpu.VMEM((CH, D), jnp.float32),
            pltpu.VMEM((ZPT, D), jnp.float32),
            pltpu.VMEM_SHARED((N_PAD, D), jnp.float32),
            pltpu.SemaphoreType.DMA,
            pltpu.SemaphoreType.DMA,
            pltpu.SemaphoreType.DMA,
            pltpu.SemaphoreType.DMA,
        ],
    )
    def k(table_hbm, row_hbm, col_hbm, w_hbm, out0, out1,
          ridx_v, cidx_v, w0_v, w1_v, rows0_v, rows1_v, ztile_v,
          accum, sem0, sem1, wsem0, wsem1):
        cid = lax.axis_index("c")
        sid = lax.axis_index("s")
        wid = cid * NS + sid
        _zero_vmem_rows(ztile_v, ZPT, D)
        pltpu.sync_copy(ztile_v, accum.at[pl.ds(sid * ZPT, ZPT)])
        pltpu.sync_copy(row_hbm.at[pl.ds(wid * NCHUNK, NCHUNK)], ridx_v)
        pltpu.sync_copy(col_hbm.at[pl.ds(wid * NCHUNK, NCHUNK)], cidx_v)
        plsc.subcore_barrier()

        rbufs = (rows0_v, rows1_v)
        sems = (sem0, sem1)
        wbufs = (w0_v, w1_v)
        wsems = (wsem0, wsem1)

        def start(i, b):
            pltpu.async_copy(table_hbm.at[ridx_v.at[i]], rbufs[b], sems[b])
            if has_w:
                pltpu.async_copy(
                    w_hbm.at[pl.ds(wid * EPT + i * CH, CH)], wbufs[b],
                    wsems[b])

        def finish(i, b):
            pltpu.make_async_copy(
                table_hbm.at[pl.ds(0, CH)], rbufs[b], sems[b]).wait()
            if has_w:
                pltpu.make_async_copy(
                    w_hbm.at[pl.ds(0, CH)], wbufs[b], wsems[b]).wait()

                def scale(e, _):
                    bc = wbufs[b][e, :]
                    for j in range(D // 16):
                        s = pl.ds(16 * j, 16)
                        rbufs[b][e, s] = rbufs[b][e, s] * bc
                    return 0
                lax.fori_loop(0, CH, scale, 0)
            pltpu.sync_copy(rbufs[b], accum.at[cidx_v.at[i]], add=True)

        start(0, 0)

        def pair(it, _):
            start(2 * it + 1, 1)
            finish(2 * it, 0)

            @pl.when(it < NCHUNK // 2 - 1)
            def _pre():
                start(2 * it + 2, 0)
            finish(2 * it + 1, 1)
            return 0
        lax.fori_loop(0, NCHUNK // 2, pair, 0)
        plsc.subcore_barrier()

        @pl.when(cid == 0)
        def _c0():
            pltpu.sync_copy(accum.at[pl.ds(sid * ZPT, ZPT)],
                            out0.at[pl.ds(sid * ZPT, ZPT)])

        @pl.when(cid == 1)
        def _c1():
            pltpu.sync_copy(accum.at[pl.ds(sid * ZPT, ZPT)],
                            out1.at[pl.ds(sid * ZPT, ZPT)])

    if not has_w:
        ewrows = jnp.zeros((E_PAD, H), jnp.float32)
    return k(table, row_pad, col_pad, ewrows)


def _sc_edge_gather(h1, row_pad, col_pad):
    @functools.partial(
        pl.kernel, mesh=_sc_mesh(), **_SC_PARAMS,
        out_type=[
            jax.ShapeDtypeStruct((E_PAD, H), jnp.float32),
            jax.ShapeDtypeStruct((E_PAD, H), jnp.float32),
        ],
        scratch_types=[
            pltpu.VMEM((2 * NCHUNK, CH), jnp.int32),
            pltpu.VMEM((CH, H), jnp.float32),
            pltpu.VMEM((CH, H), jnp.float32),
            pltpu.SemaphoreType.DMA,
            pltpu.SemaphoreType.DMA,
        ],
    )
    def k(h1_hbm, row_hbm, col_hbm, xi_hbm, xj_hbm, idx_v, rows0_v, rows1_v,
          sem0, sem1):
        cid = lax.axis_index("c")
        sid = lax.axis_index("s")
        wid = cid * NS + sid
        pltpu.sync_copy(row_hbm.at[pl.ds(wid * NCHUNK, NCHUNK)],
                        idx_v.at[pl.ds(0, NCHUNK)])
        pltpu.sync_copy(col_hbm.at[pl.ds(wid * NCHUNK, NCHUNK)],
                        idx_v.at[pl.ds(NCHUNK, NCHUNK)])

        def shift(r, _):
            for t in range(CH // 16):
                s = pl.ds(16 * t, 16)
                v = idx_v[r, s] - 1
                idx_v[r, s] = jnp.where(v < 0, v + N, v)
            return 0
        lax.fori_loop(0, 2 * NCHUNK, shift, 0)

        rbufs = (rows0_v, rows1_v)
        sems = (sem0, sem1)

        def start(i, b):
            pltpu.async_copy(h1_hbm.at[idx_v.at[i]], rbufs[b], sems[b])

        def finish(i, b):
            pltpu.make_async_copy(
                h1_hbm.at[pl.ds(0, CH)], rbufs[b], sems[b]).wait()
            half = i // NCHUNK
            j = i - half * NCHUNK
            base = wid * EPT + j * CH

            @pl.when(half == 0)
            def _xi():
                pltpu.sync_copy(rbufs[b], xi_hbm.at[pl.ds(base, CH)])

            @pl.when(half == 1)
            def _xj():
                pltpu.sync_copy(rbufs[b], xj_hbm.at[pl.ds(base, CH)])

        start(0, 0)

        def pair(it, _):
            start(2 * it + 1, 1)
            finish(2 * it, 0)

            @pl.when(it < NCHUNK - 1)
            def _pre():
                start(2 * it + 2, 0)
            finish(2 * it + 1, 1)
            return 0
        lax.fori_loop(0, NCHUNK, pair, 0)

    return k(h1, row_pad, col_pad)


def _sc_root_gather(x0, h1, root_index, batch_pad):
    @functools.partial(
        pl.kernel, mesh=_sc_mesh(), **_SC_PARAMS,
        out_type=[
            jax.ShapeDtypeStruct((N_PAD, D_IN), jnp.float32),
            jax.ShapeDtypeStruct((N_PAD, H), jnp.float32),
        ],
        scratch_types=[
            pltpu.VMEM((RCH,), jnp.int32),
            pltpu.VMEM((RCH,), jnp.int32),
            pltpu.VMEM((RCH, D_IN), jnp.float32),
            pltpu.VMEM((RCH, H), jnp.float32),
            pltpu.SemaphoreType.DMA,
        ],
    )
    def k(x0_hbm, h1_hbm, ri_hbm, bat_hbm, re_hbm, re2_hbm,
          braw_v, idx_v, rows1_v, rows2_v, sem):
        cid = lax.axis_index("c")
        sid = lax.axis_index("s")
        wid = cid * NS + sid

        def chunk(i, _):
            base = wid * NPT + i * RCH
            pltpu.sync_copy(bat_hbm.at[pl.ds(base, RCH)], braw_v)
            pltpu.async_copy(ri_hbm.at[braw_v], idx_v, sem).wait()
            pltpu.async_copy(x0_hbm.at[idx_v], rows1_v, sem).wait()
            pltpu.sync_copy(rows1_v, re_hbm.at[pl.ds(base, RCH)])
            pltpu.async_copy(h1_hbm.at[idx_v], rows2_v, sem).wait()
            pltpu.sync_copy(rows2_v, re2_hbm.at[pl.ds(base, RCH)])
            return 0
        lax.fori_loop(0, NPT // RCH, chunk, 0)

    return k(x0, h1, root_index, batch_pad)


def kernel(node_features, edge_index, root_index, batch_size, params, noise):
    x0 = node_features
    row, col = edge_index[0], edge_index[1]

    npad = E_PAD - E
    pad_src = (jnp.arange(npad, dtype=jnp.int32) % 240)
    pad_dst = N + (jnp.arange(npad, dtype=jnp.int32) % (N_PAD - N))
    row2d = jnp.concatenate([row, pad_src]).reshape(E_PAD // CH, CH)
    col2d = jnp.concatenate([col, pad_dst]).reshape(E_PAD // CH, CH)
    batch_pad = jnp.concatenate(
        [batch_size, jnp.zeros((N_PAD - N,), jnp.int32)])

    d1p0, d1p1 = _sc_hist(col2d)
    xws1, dis1 = _tc_xw1(x0, params['W1'], d1p0, d1p1)

    c1p0, c1p1 = _sc_conv_scatter(xws1, row2d, col2d, H)
    h1 = _tc_h1(c1p0, c1p1, xws1, dis1, params['b1'])

    xi, xj = _sc_edge_gather(h1, row2d, col2d)
    root_ext, root_ext2 = _sc_root_gather(x0, h1, root_index, batch_pad)

    xiT = xi[:E].T
    xjT = xj[:E].T
    noiseT = noise[:, 0, :].T

    S17 = _tc_moments(xiT, xjT)
    cntm = float(E * H)
    Mm = S17[H, :H] / cntm
    Sm = S17[:H, :H] / cntm
    w1e_rows, be_rows, w2b_rows, b2s_rows = [], [], [], []
    for k, name in enumerate(('sim', 'wm', 'wb', 'bm', 'bb')):
        p = params[name]
        w1 = p['w1']
        mean = w1 @ Mm
        ey2 = jnp.einsum('oc,cd,od->o', w1, Sm, w1)
        var = ey2 - mean * mean
        sc = p['g'] * lax.rsqrt(var + EPS)
        w1e_rows.append(w1 * sc[:, None])
        be_rows.append(p['b'] - mean * sc)
        wrow = jnp.zeros((5, 80), jnp.float32).at[k, 16 * k:16 * k + 16].set(p['w2'][0])
        w2b_rows.append(wrow)
        b2s_rows.append(p['b2'][0])
    W1eff = jnp.concatenate(w1e_rows, axis=0)
    beff = jnp.concatenate(be_rows).reshape(80, 1)
    W1eff17 = jnp.concatenate([W1eff, beff], axis=1)
    W2blk = sum(w2b_rows)
    b2s = jnp.stack(b2s_rows).reshape(5, 1)

    ep3, klsum = _tc_pass2(xiT, xjT, noiseT, W1eff17, W2blk, b2s,
                           params['fc1'], params['fc2'])
    ep = ep3.reshape(E)
    edge_loss = klsum[0, 0] / float(E)

    s1, q1 = _tc_bnstat(h1, root_ext)
    mb = s1 / float(N)
    vb = q1 / float(N) - mb * mb
    scb = params['bn1_g'].reshape(1, -1) * lax.rsqrt(vb + EPS)
    shb = params['bn1_b'].reshape(1, -1) - mb * scb

    ep_rows = jnp.concatenate(
        [jnp.broadcast_to(ep[:, None], (E, H)),
         jnp.zeros((E_PAD - E, H), jnp.float32)])
    d2p0, d2p1 = _sc_hist(col2d, ep_rows)
    xws2, dis2 = _tc_xw2(h1, root_ext, scb, shb, params['W2'], d2p0, d2p1)
    HD = D_IN // 2
    c2p0a, c2p1a = _sc_conv_scatter(xws2[:, :HD], row2d, col2d, HD,
                                    ewrows=ep_rows)
    c2p0b, c2p1b = _sc_conv_scatter(xws2[:, HD:], row2d, col2d, HD,
                                    ewrows=ep_rows)

    batch3 = batch_size.reshape(NRB, 1, RB)
    out = _tc_out(c2p0a, c2p1a, c2p0b, c2p1b, xws2, dis2, params['b2'],
                  root_ext2, batch3)
    return out, edge_loss


# speedup vs baseline: 10.0810x; 1.0155x over previous
import functools

import jax
import jax.numpy as jnp
import numpy as np
from jax import lax
from jax.experimental import pallas as pl
from jax.experimental.pallas import tpu as pltpu

N = 10000
E = 160000
D_IN = 128
H = 16
G = 64
EPS = 1e-5

_INTERP = False

EB = 640
NEB = E // EB
RB = 1000
NRB = N // RB


def _xw1_body(x0_ref, w1_ref, p0_ref, p1_ref, xws_ref, dis_ref):
    dis = lax.rsqrt(p0_ref[...] + p1_ref[...] + 1.0)
    xw = jnp.dot(x0_ref[...], w1_ref[...], preferred_element_type=jnp.float32)
    xws_ref[...] = xw * dis
    dis_ref[...] = dis


def _tc_xw1(x0, W1, p0, p1):
    return pl.pallas_call(
        _xw1_body,
        grid=(NRB,),
        in_specs=[
            pl.BlockSpec((RB, D_IN), lambda i: (i, 0)),
            pl.BlockSpec((D_IN, H), lambda i: (0, 0)),
            pl.BlockSpec((RB, H), lambda i: (i, 0)),
            pl.BlockSpec((RB, H), lambda i: (i, 0)),
        ],
        out_specs=[
            pl.BlockSpec((RB, H), lambda i: (i, 0)),
            pl.BlockSpec((RB, H), lambda i: (i, 0)),
        ],
        out_shape=[
            jax.ShapeDtypeStruct((N, H), jnp.float32),
            jax.ShapeDtypeStruct((N, H), jnp.float32),
        ],
        interpret=_INTERP,
    )(x0, W1, p0, p1)


def _h1_body(p0_ref, p1_ref, xws_ref, dis_ref, b_ref, o_ref):
    o_ref[...] = dis_ref[...] * (p0_ref[...] + p1_ref[...] + xws_ref[...]) + b_ref[...]


def _tc_h1(p0, p1, xws1, dis1, b1):
    return pl.pallas_call(
        _h1_body,
        grid=(NRB,),
        in_specs=[
            pl.BlockSpec((RB, H), lambda i: (i, 0)),
            pl.BlockSpec((RB, H), lambda i: (i, 0)),
            pl.BlockSpec((RB, H), lambda i: (i, 0)),
            pl.BlockSpec((RB, H), lambda i: (i, 0)),
            pl.BlockSpec((1, H), lambda i: (0, 0)),
        ],
        out_specs=pl.BlockSpec((RB, H), lambda i: (i, 0)),
        out_shape=jax.ShapeDtypeStruct((N, H), jnp.float32),
        interpret=_INTERP,
    )(p0, p1, xws1, dis1, b1.reshape(1, H))


def _build_z17(zb_s, xiT, xjT):
    for l in range(H):
        zb_s[0:H, l * EB:(l + 1) * EB] = jnp.abs(xiT - xjT[l:l + 1, :])


def _moments_body(xiT_ref, xjT_ref, s_ref, zb_s):
    @pl.when(pl.program_id(0) == 0)
    def _init():
        s_ref[...] = jnp.zeros_like(s_ref)
        zb_s[H:H + 1, :] = jnp.ones((1, H * EB), jnp.float32)

    _build_z17(zb_s, xiT_ref[...], xjT_ref[...])
    zb = zb_s[...]
    s_ref[...] += lax.dot_general(zb, zb, (((1,), (1,)), ((), ())),
                                  preferred_element_type=jnp.float32)


def _tc_moments(xiT, xjT):
    return pl.pallas_call(
        _moments_body,
        grid=(NEB,),
        in_specs=[
            pl.BlockSpec((H, EB), lambda i: (0, i)),
            pl.BlockSpec((H, EB), lambda i: (0, i)),
        ],
        out_specs=pl.BlockSpec((H + 1, H + 1), lambda i: (0, 0)),
        out_shape=jax.ShapeDtypeStruct((H + 1, H + 1), jnp.float32),
        scratch_shapes=[pltpu.VMEM((H + 1, H * EB), jnp.float32)],
        interpret=_INTERP,
    )(xiT, xjT)


def _logsumexp0(x):
    mx = jnp.max(x, axis=0, keepdims=True)
    return jnp.log(jnp.sum(jnp.exp(x - mx), axis=0, keepdims=True)) + mx


def _pass2_body(xiT_ref, xjT_ref, nzT_ref, w1e_ref, w2b_ref, b2s_ref,
                fc1_ref, fc2_ref, ep_ref, kl_ref, zb_s,
                sim_s, wm_s, wb_s, bm_s, bb_s):
    @pl.when(pl.program_id(0) == 0)
    def _init():
        kl_ref[...] = jnp.zeros_like(kl_ref)

    @pl.when(pl.program_id(0) == 0)
    def _ones():
        zb_s[H:H + 1, :] = jnp.ones((1, H * EB), jnp.float32)

    _build_z17(zb_s, xiT_ref[...], xjT_ref[...])
    w2b = w2b_ref[...]
    b2s = b2s_ref[...]
    y = jnp.dot(w1e_ref[...], zb_s[...],
                preferred_element_type=jnp.float32)
    y = jnp.where(y > 0, y, 0.01 * y)
    for l in range(H):
        s5 = jnp.dot(w2b, y[:, l * EB:(l + 1) * EB],
                     preferred_element_type=jnp.float32) + b2s
        sim_s[l:l + 1, :] = s5[0:1, :]
        wm_s[l:l + 1, :] = s5[1:2, :]
        wb_s[l:l + 1, :] = s5[2:3, :]
        bm_s[l:l + 1, :] = s5[3:4, :]
        bb_s[l:l + 1, :] = s5[4:5, :]
    sv = sim_s[...]
    ep_logits = jnp.dot(fc1_ref[...], sv, preferred_element_type=jnp.float32)
    edge_pred = jax.nn.sigmoid(ep_logits)
    lm = wm_s[...] * sv + bm_s[...]
    lv = jnp.abs(jnp.log(sv * sv * jnp.exp(wb_s[...]) + jnp.exp(bb_s[...])))
    ey_in = jax.nn.sigmoid(lm + lv * nzT_ref[...])
    edge_y = jnp.dot(fc2_ref[...], ey_in, preferred_element_type=jnp.float32)
    logp_x = edge_pred - _logsumexp0(edge_pred)
    logp_y = edge_y - _logsumexp0(edge_y)
    p_y = jnp.exp(logp_y)
    kl_blk = jnp.sum(jnp.sum(p_y * (logp_y - logp_x), axis=1, keepdims=True),
                     axis=0, keepdims=True)
    kl_ref[...] += kl_blk
    ep_ref[...] = jnp.mean(edge_pred, axis=0, keepdims=True)[None]


def _tc_pass2(xiT, xjT, noiseT, W1eff17, W2blk, b2s, fc1, fc2):
    return pl.pallas_call(
        _pass2_body,
        grid=(NEB,),
        in_specs=[
            pl.BlockSpec((H, EB), lambda i: (0, i)),
            pl.BlockSpec((H, EB), lambda i: (0, i)),
            pl.BlockSpec((H, EB), lambda i: (0, i)),
            pl.BlockSpec((80, H + 1), lambda i: (0, 0)),
            pl.BlockSpec((5, 80), lambda i: (0, 0)),
            pl.BlockSpec((5, 1), lambda i: (0, 0)),
            pl.BlockSpec((4, H), lambda i: (0, 0)),
            pl.BlockSpec((4, H), lambda i: (0, 0)),
        ],
        out_specs=[
            pl.BlockSpec((1, 1, EB), lambda i: (i, 0, 0)),
            pl.BlockSpec((1, 1), lambda i: (0, 0)),
        ],
        out_shape=[
            jax.ShapeDtypeStruct((NEB, 1, EB), jnp.float32),
            jax.ShapeDtypeStruct((1, 1), jnp.float32),
        ],
        scratch_shapes=[pltpu.VMEM((H + 1, H * EB), jnp.float32)]
        + [pltpu.VMEM((H, EB), jnp.float32) for _ in range(5)],
        interpret=_INTERP,
    )(xiT, xjT, noiseT, W1eff17, W2blk, b2s, fc1, fc2)


def _bnstat_body(h1_ref, re_ref, s_ref, q_ref):
    @pl.when(pl.program_id(0) == 0)
    def _init():
        s_ref[...] = jnp.zeros_like(s_ref)
        q_ref[...] = jnp.zeros_like(q_ref)

    x = jnp.concatenate([h1_ref[...], re_ref[...]], axis=1)
    s_ref[...] += jnp.sum(x, axis=0, keepdims=True)
    q_ref[...] += jnp.sum(x * x, axis=0, keepdims=True)


def _tc_bnstat(h1, root_ext):
    return pl.pallas_call(
        _bnstat_body,
        grid=(NRB,),
        in_specs=[
            pl.BlockSpec((RB, H), lambda i: (i, 0)),
            pl.BlockSpec((RB, D_IN), lambda i: (i, 0)),
        ],
        out_specs=[
            pl.BlockSpec((1, H + D_IN), lambda i: (0, 0)),
            pl.BlockSpec((1, H + D_IN), lambda i: (0, 0)),
        ],
        out_shape=[
            jax.ShapeDtypeStruct((1, H + D_IN), jnp.float32),
            jax.ShapeDtypeStruct((1, H + D_IN), jnp.float32),
        ],
        interpret=_INTERP,
    )(h1, root_ext)


def _xw2_body(h1_ref, re_ref, sc_ref, sh_ref, w2_ref, p0_ref, p1_ref,
              xws_ref, dis_ref):
    dis = lax.rsqrt(p0_ref[...] + p1_ref[...] + 1.0)
    x = jnp.concatenate([h1_ref[...], re_ref[...]], axis=1)
    x = jax.nn.relu(x * sc_ref[...] + sh_ref[...])
    xw = jnp.dot(x, w2_ref[...], preferred_element_type=jnp.float32)
    xws_ref[...] = xw * dis[:, 0:1]
    dis_ref[...] = dis


def _tc_xw2(h1, root_ext, scale, shift, W2, p0, p1):
    return pl.pallas_call(
        _xw2_body,
        grid=(NRB,),
        in_specs=[
            pl.BlockSpec((RB, H), lambda i: (i, 0)),
            pl.BlockSpec((RB, D_IN), lambda i: (i, 0)),
            pl.BlockSpec((1, H + D_IN), lambda i: (0, 0)),
            pl.BlockSpec((1, H + D_IN), lambda i: (0, 0)),
            pl.BlockSpec((H + D_IN, D_IN), lambda i: (0, 0)),
            pl.BlockSpec((RB, H), lambda i: (i, 0)),
            pl.BlockSpec((RB, H), lambda i: (i, 0)),
        ],
        out_specs=[
            pl.BlockSpec((RB, D_IN), lambda i: (i, 0)),
            pl.BlockSpec((RB, H), lambda i: (i, 0)),
        ],
        out_shape=[
            jax.ShapeDtypeStruct((N, D_IN), jnp.float32),
            jax.ShapeDtypeStruct((N, H), jnp.float32),
        ],
        interpret=_INTERP,
    )(h1, root_ext, scale, shift, W2, p0, p1)


def _out_body(p0a_ref, p1a_ref, p0b_ref, p1b_ref, xws_ref, dis_ref, b2_ref,
              re2_ref, bat_ref, out_ref, sums, cnts):
    @pl.when(pl.program_id(0) == 0)
    def _init():
        sums[...] = jnp.zeros_like(sums)
        cnts[...] = jnp.zeros_like(cnts)

    psum = jnp.concatenate([p0a_ref[...] + p1a_ref[...],
                            p0b_ref[...] + p1b_ref[...]], axis=1)
    x2 = jax.nn.relu(dis_ref[...][:, 0:1]
                     * (psum + xws_ref[...]) + b2_ref[...])
    z = jnp.concatenate([x2, re2_ref[...]], axis=1)
    bat = bat_ref[0]
    oh = (lax.broadcasted_iota(jnp.int32, (G, RB), 0) == bat).astype(jnp.float32)
    sums[...] += jnp.dot(oh, z, preferred_element_type=jnp.float32)
    cnts[...] += jnp.sum(oh, axis=1, keepdims=True)

    @pl.when(pl.program_id(0) == NRB - 1)
    def _fin():
        out_ref[...] = sums[...] / jnp.maximum(cnts[...], 1.0)


def _tc_out(p0a, p1a, p0b, p1b, xws2, dis2, b2, root_ext2, batch3):
    HD = D_IN // 2
    return pl.pallas_call(
        _out_body,
        grid=(NRB,),
        in_specs=[
            pl.BlockSpec((RB, HD), lambda i: (i, 0)),
            pl.BlockSpec((RB, HD), lambda i: (i, 0)),
            pl.BlockSpec((RB, HD), lambda i: (i, 0)),
            pl.BlockSpec((RB, HD), lambda i: (i, 0)),
            pl.BlockSpec((RB, D_IN), lambda i: (i, 0)),
            pl.BlockSpec((RB, H), lambda i: (i, 0)),
            pl.BlockSpec((1, D_IN), lambda i: (0, 0)),
            pl.BlockSpec((RB, H), lambda i: (i, 0)),
            pl.BlockSpec((1, 1, RB), lambda i: (i, 0, 0)),
        ],
        out_specs=pl.BlockSpec((G, H + D_IN), lambda i: (0, 0)),
        out_shape=jax.ShapeDtypeStruct((G, H + D_IN), jnp.float32),
        scratch_shapes=[
            pltpu.VMEM((G, H + D_IN), jnp.float32),
            pltpu.VMEM((G, 1), jnp.float32),
        ],
        interpret=_INTERP,
    )(p0a, p1a, p0b, p1b, xws2, dis2, b2.reshape(1, D_IN), root_ext2, batch3)


from jax.experimental.pallas import tpu_sc as plsc

NC = 2
NS = 16
NW = NC * NS
CH = 128
E_PAD = 163840
EPT = E_PAD // NW
NCHUNK = EPT // CH
N_PAD = 10240
ZPT = N_PAD // NS
NPT = N_PAD // NW
RCH = 64


def _sc_mesh():
    return plsc.VectorSubcoreMesh(core_axis_name="c", subcore_axis_name="s")


_SC_PARAMS = dict(compiler_params=pltpu.CompilerParams(use_tc_tiling_on_sc=False))


def _zero_vmem_rows(ref, nrows, width):
    def zrow(r, _):
        for j in range(width // 16):
            ref[r, pl.ds(16 * j, 16)] = jnp.zeros((16,), jnp.float32)
        return 0
    lax.fori_loop(0, nrows, zrow, 0)


def _sc_hist(col2d, wrows_pad=None):
    has_w = wrows_pad is not None

    @functools.partial(
        pl.kernel, mesh=_sc_mesh(), **_SC_PARAMS,
        out_type=[
            jax.ShapeDtypeStruct((N_PAD, H), jnp.float32),
            jax.ShapeDtypeStruct((N_PAD, H), jnp.float32),
        ],
        scratch_types=[
            pltpu.VMEM((NCHUNK, CH), jnp.int32),
            pltpu.VMEM((CH, H), jnp.float32),
            pltpu.VMEM((CH, H), jnp.float32),
            pltpu.VMEM((ZPT, H), jnp.float32),
            pltpu.VMEM_SHARED((N_PAD, H), jnp.float32),
            pltpu.SemaphoreType.DMA,
            pltpu.SemaphoreType.DMA,
        ],
    )
    def k(col_hbm, w_hbm, out0, out1, cidx_v, rows0_v, rows1_v, ztile_v,
          accum, sem0, sem1):
        cid = lax.axis_index("c")
        sid = lax.axis_index("s")
        wid = cid * NS + sid
        _zero_vmem_rows(ztile_v, ZPT, H)
        pltpu.sync_copy(ztile_v, accum.at[pl.ds(sid * ZPT, ZPT)])
        pltpu.sync_copy(col_hbm.at[pl.ds(wid * NCHUNK, NCHUNK)], cidx_v)
        if not has_w:
            def onerow(r, _):
                rows0_v[r, :] = jnp.ones((16,), jnp.float32)
                return 0
            lax.fori_loop(0, CH, onerow, 0)
        plsc.subcore_barrier()

        if has_w:
            bufs = (rows0_v, rows1_v)
            sems = (sem0, sem1)

            def start(i, b):
                pltpu.async_copy(
                    w_hbm.at[pl.ds(wid * EPT + i * CH, CH)], bufs[b], sems[b])

            start(0, 0)

            def pair(it, _):
                pltpu.async_copy(
                    w_hbm.at[pl.ds(wid * EPT + (2 * it + 1) * CH, CH)],
                    rows1_v, sem1)
                pltpu.make_async_copy(
                    w_hbm.at[pl.ds(0, CH)], rows0_v, sem0).wait()
                pltpu.sync_copy(rows0_v, accum.at[cidx_v.at[2 * it]], add=True)

                @pl.when(it < NCHUNK // 2 - 1)
                def _pre():
                    pltpu.async_copy(
                        w_hbm.at[pl.ds(wid * EPT + (2 * it + 2) * CH, CH)],
                        rows0_v, sem0)
                pltpu.make_async_copy(
                    w_hbm.at[pl.ds(0, CH)], rows1_v, sem1).wait()
                pltpu.sync_copy(rows1_v, accum.at[cidx_v.at[2 * it + 1]],
                                add=True)
                return 0
            lax.fori_loop(0, NCHUNK // 2, pair, 0)
        else:
            def chunk(i, _):
                pltpu.sync_copy(rows0_v, accum.at[cidx_v.at[i]], add=True)
                return 0
            lax.fori_loop(0, NCHUNK, chunk, 0)
        plsc.subcore_barrier()

        @pl.when(cid == 0)
        def _c0():
            pltpu.sync_copy(accum.at[pl.ds(sid * ZPT, ZPT)],
                            out0.at[pl.ds(sid * ZPT, ZPT)])

        @pl.when(cid == 1)
        def _c1():
            pltpu.sync_copy(accum.at[pl.ds(sid * ZPT, ZPT)],
                            out1.at[pl.ds(sid * ZPT, ZPT)])

    if not has_w:
        wrows_pad = jnp.zeros((8, H), jnp.float32)
    return k(col2d, wrows_pad)


def _sc_conv_scatter(table, row_pad, col_pad, D, ewrows=None):
    has_w = ewrows is not None

    @functools.partial(
        pl.kernel, mesh=_sc_mesh(), **_SC_PARAMS,
        out_type=[
            jax.ShapeDtypeStruct((N_PAD, D), jnp.float32),
            jax.ShapeDtypeStruct((N_PAD, D), jnp.float32),
        ],
        scratch_types=[
            pltpu.VMEM((NCHUNK, CH), jnp.int32),
            pltpu.VMEM((NCHUNK, CH), jnp.int32),
            pltpu.VMEM((CH, H), jnp.float32),
            pltpu.VMEM((CH, H), jnp.float32),
            pltpu.VMEM((CH, H), jnp.float32),
            pltpu.VMEM((CH, H), jnp.float32),
            pltpu.VMEM((CH, D), jnp.float32),
            pltpu.VMEM((CH, D), jnp.float32),
            pltpu.VMEM((CH, D), jnp.float32),
            pltpu.VMEM((CH, D), jnp.float32),
            pltpu.VMEM_SHARED((N_PAD, D), jnp.float32),
        ] + [pltpu.SemaphoreType.DMA] * 12,
    )
    def k(table_hbm, row_hbm, col_hbm, w_hbm, out0, out1,
          ridx_v, cidx_v, w0_v, w1_v, w2_v, w3_v, r0_v, r1_v, r2_v, r3_v,
          accum, g0, g1, g2, g3, s0, s1, s2, s3, wg0, wg1, wg2, wg3):
        cid = lax.axis_index("c")
        sid = lax.axis_index("s")
        wid = cid * NS + sid
        _zero_vmem_rows(r0_v, CH, D)
        for zz in range(ZPT // CH):
            pltpu.sync_copy(r0_v, accum.at[pl.ds(sid * ZPT + zz * CH, CH)])
        pltpu.sync_copy(row_hbm.at[pl.ds(wid * NCHUNK, NCHUNK)], ridx_v)
        pltpu.sync_copy(col_hbm.at[pl.ds(wid * NCHUNK, NCHUNK)], cidx_v)
        plsc.subcore_barrier()

        rbufs = (r0_v, r1_v, r2_v, r3_v)
        gsems = (g0, g1, g2, g3)
        ssems = (s0, s1, s2, s3)
        wbufs = (w0_v, w1_v, w2_v, w3_v)
        wsems = (wg0, wg1, wg2, wg3)

        def gstart(i, b, wb):
            pltpu.async_copy(table_hbm.at[ridx_v.at[i]], rbufs[b], gsems[b])
            if has_w:
                pltpu.async_copy(
                    w_hbm.at[pl.ds(wid * EPT + i * CH, CH)], wbufs[wb],
                    wsems[wb])

        def process(i, b, wb):
            pltpu.make_async_copy(
                table_hbm.at[pl.ds(0, CH)], rbufs[b], gsems[b]).wait()
            if has_w:
                pltpu.make_async_copy(
                    w_hbm.at[pl.ds(0, CH)], wbufs[wb], wsems[wb]).wait()

                def scale(e2, _):
                    for e in (2 * e2, 2 * e2 + 1):
                        bc = wbufs[wb][e, :]
                        for j in range(D // 16):
                            s = pl.ds(16 * j, 16)
                            rbufs[b][e, s] = rbufs[b][e, s] * bc
                    return 0
                lax.fori_loop(0, CH // 2, scale, 0)
            pltpu.sync_copy(rbufs[b], accum.at[cidx_v.at[i]], add=True)

        gstart(0, 0, 0)
        gstart(1, 1, 1)

        def group(it, _):
            for kk in range(4):
                i = 4 * it + kk
                b = kk
                wb = kk
                nxt = i + 2
                nb = (kk + 2) % 4

                @pl.when(nxt < NCHUNK)
                def _pre():
                    gstart(nxt, nb, nb)
                process(i, b, wb)
            return 0
        lax.fori_loop(0, NCHUNK // 4, group, 0)
        plsc.subcore_barrier()

        @pl.when(cid == 0)
        def _c0():
            pltpu.sync_copy(accum.at[pl.ds(sid * ZPT, ZPT)],
                            out0.at[pl.ds(sid * ZPT, ZPT)])

        @pl.when(cid == 1)
        def _c1():
            pltpu.sync_copy(accum.at[pl.ds(sid * ZPT, ZPT)],
                            out1.at[pl.ds(sid * ZPT, ZPT)])

    if not has_w:
        ewrows = jnp.zeros((E_PAD, H), jnp.float32)
    return k(table, row_pad, col_pad, ewrows)


def _sc_edge_gather(h1, row_pad, col_pad):
    @functools.partial(
        pl.kernel, mesh=_sc_mesh(), **_SC_PARAMS,
        out_type=[
            jax.ShapeDtypeStruct((E_PAD, H), jnp.float32),
            jax.ShapeDtypeStruct((E_PAD, H), jnp.float32),
        ],
        scratch_types=[
            pltpu.VMEM((2 * NCHUNK, CH), jnp.int32),
            pltpu.VMEM((CH, H), jnp.float32),
            pltpu.VMEM((CH, H), jnp.float32),
            pltpu.SemaphoreType.DMA,
            pltpu.SemaphoreType.DMA,
        ],
    )
    def k(h1_hbm, row_hbm, col_hbm, xi_hbm, xj_hbm, idx_v, rows0_v, rows1_v,
          sem0, sem1):
        cid = lax.axis_index("c")
        sid = lax.axis_index("s")
        wid = cid * NS + sid
        pltpu.sync_copy(row_hbm.at[pl.ds(wid * NCHUNK, NCHUNK)],
                        idx_v.at[pl.ds(0, NCHUNK)])
        pltpu.sync_copy(col_hbm.at[pl.ds(wid * NCHUNK, NCHUNK)],
                        idx_v.at[pl.ds(NCHUNK, NCHUNK)])

        def shift(r, _):
            for t in range(CH // 16):
                s = pl.ds(16 * t, 16)
                v = idx_v[r, s] - 1
                idx_v[r, s] = jnp.where(v < 0, v + N, v)
            return 0
        lax.fori_loop(0, 2 * NCHUNK, shift, 0)

        rbufs = (rows0_v, rows1_v)
        sems = (sem0, sem1)

        def start(i, b):
            pltpu.async_copy(h1_hbm.at[idx_v.at[i]], rbufs[b], sems[b])

        def finish(i, b):
            pltpu.make_async_copy(
                h1_hbm.at[pl.ds(0, CH)], rbufs[b], sems[b]).wait()
            half = i // NCHUNK
            j = i - half * NCHUNK
            base = wid * EPT + j * CH

            @pl.when(half == 0)
            def _xi():
                pltpu.sync_copy(rbufs[b], xi_hbm.at[pl.ds(base, CH)])

            @pl.when(half == 1)
            def _xj():
                pltpu.sync_copy(rbufs[b], xj_hbm.at[pl.ds(base, CH)])

        start(0, 0)

        def pair(it, _):
            start(2 * it + 1, 1)
            finish(2 * it, 0)

            @pl.when(it < NCHUNK - 1)
            def _pre():
                start(2 * it + 2, 0)
            finish(2 * it + 1, 1)
            return 0
        lax.fori_loop(0, NCHUNK, pair, 0)

    return k(h1, row_pad, col_pad)


def _sc_root_gather(x0, h1, root_index, batch_pad):
    @functools.partial(
        pl.kernel, mesh=_sc_mesh(), **_SC_PARAMS,
        out_type=[
            jax.ShapeDtypeStruct((N_PAD, D_IN), jnp.float32),
            jax.ShapeDtypeStruct((N_PAD, H), jnp.float32),
        ],
        scratch_types=[
            pltpu.VMEM((RCH,), jnp.int32),
            pltpu.VMEM((RCH,), jnp.int32),
            pltpu.VMEM((RCH, D_IN), jnp.float32),
            pltpu.VMEM((RCH, H), jnp.float32),
            pltpu.SemaphoreType.DMA,
        ],
    )
    def k(x0_hbm, h1_hbm, ri_hbm, bat_hbm, re_hbm, re2_hbm,
          braw_v, idx_v, rows1_v, rows2_v, sem):
        cid = lax.axis_index("c")
        sid = lax.axis_index("s")
        wid = cid * NS + sid

        def chunk(i, _):
            base = wid * NPT + i * RCH
            pltpu.sync_copy(bat_hbm.at[pl.ds(base, RCH)], braw_v)
            pltpu.async_copy(ri_hbm.at[braw_v], idx_v, sem).wait()
            pltpu.async_copy(x0_hbm.at[idx_v], rows1_v, sem).wait()
            pltpu.sync_copy(rows1_v, re_hbm.at[pl.ds(base, RCH)])
            pltpu.async_copy(h1_hbm.at[idx_v], rows2_v, sem).wait()
            pltpu.sync_copy(rows2_v, re2_hbm.at[pl.ds(base, RCH)])
            return 0
        lax.fori_loop(0, NPT // RCH, chunk, 0)

    return k(x0, h1, root_index, batch_pad)


def kernel(node_features, edge_index, root_index, batch_size, params, noise):
    x0 = node_features
    row, col = edge_index[0], edge_index[1]

    npad = E_PAD - E
    pad_src = (jnp.arange(npad, dtype=jnp.int32) % 240)
    pad_dst = N + (jnp.arange(npad, dtype=jnp.int32) % (N_PAD - N))
    row2d = jnp.concatenate([row, pad_src]).reshape(E_PAD // CH, CH)
    col2d = jnp.concatenate([col, pad_dst]).reshape(E_PAD // CH, CH)
    batch_pad = jnp.concatenate(
        [batch_size, jnp.zeros((N_PAD - N,), jnp.int32)])

    d1p0, d1p1 = _sc_hist(col2d)
    xws1, dis1 = _tc_xw1(x0, params['W1'], d1p0, d1p1)

    c1p0, c1p1 = _sc_conv_scatter(xws1, row2d, col2d, H)
    h1 = _tc_h1(c1p0, c1p1, xws1, dis1, params['b1'])

    xi, xj = _sc_edge_gather(h1, row2d, col2d)
    root_ext, root_ext2 = _sc_root_gather(x0, h1, root_index, batch_pad)

    xiT = xi[:E].T
    xjT = xj[:E].T
    noiseT = noise[:, 0, :].T

    S17 = _tc_moments(xiT, xjT)
    cntm = float(E * H)
    Mm = S17[H, :H] / cntm
    Sm = S17[:H, :H] / cntm
    w1e_rows, be_rows, w2b_rows, b2s_rows = [], [], [], []
    for k, name in enumerate(('sim', 'wm', 'wb', 'bm', 'bb')):
        p = params[name]
        w1 = p['w1']
        mean = w1 @ Mm
        ey2 = jnp.einsum('oc,cd,od->o', w1, Sm, w1)
        var = ey2 - mean * mean
        sc = p['g'] * lax.rsqrt(var + EPS)
        w1e_rows.append(w1 * sc[:, None])
        be_rows.append(p['b'] - mean * sc)
        wrow = jnp.zeros((5, 80), jnp.float32).at[k, 16 * k:16 * k + 16].set(p['w2'][0])
        w2b_rows.append(wrow)
        b2s_rows.append(p['b2'][0])
    W1eff = jnp.concatenate(w1e_rows, axis=0)
    beff = jnp.concatenate(be_rows).reshape(80, 1)
    W1eff17 = jnp.concatenate([W1eff, beff], axis=1)
    W2blk = sum(w2b_rows)
    b2s = jnp.stack(b2s_rows).reshape(5, 1)

    ep3, klsum = _tc_pass2(xiT, xjT, noiseT, W1eff17, W2blk, b2s,
                           params['fc1'], params['fc2'])
    ep = ep3.reshape(E)
    edge_loss = klsum[0, 0] / float(E)

    s1, q1 = _tc_bnstat(h1, root_ext)
    mb = s1 / float(N)
    vb = q1 / float(N) - mb * mb
    scb = params['bn1_g'].reshape(1, -1) * lax.rsqrt(vb + EPS)
    shb = params['bn1_b'].reshape(1, -1) - mb * scb

    ep_rows = jnp.concatenate(
        [jnp.broadcast_to(ep[:, None], (E, H)),
         jnp.zeros((E_PAD - E, H), jnp.float32)])
    d2p0, d2p1 = _sc_hist(col2d, ep_rows)
    xws2, dis2 = _tc_xw2(h1, root_ext, scb, shb, params['W2'], d2p0, d2p1)
    HD = D_IN // 2
    c2p0a, c2p1a = _sc_conv_scatter(xws2[:, :HD], row2d, col2d, HD,
                                    ewrows=ep_rows)
    c2p0b, c2p1b = _sc_conv_scatter(xws2[:, HD:], row2d, col2d, HD,
                                    ewrows=ep_rows)

    batch3 = batch_size.reshape(NRB, 1, RB)
    out = _tc_out(c2p0a, c2p1a, c2p0b, c2p1b, xws2, dis2, params['b2'],
                  root_ext2, batch3)
    return out, edge_loss
